# Initial kernel scaffold; baseline (speedup 1.0000x reference)
import functools
import math

import jax
import jax.numpy as jnp
from jax import lax
from jax.experimental import pallas as pl
from jax.experimental.pallas import tpu as pltpu
from jax.experimental.pallas import tpu_sc as plsc

_K = 4
_N = 10000
_NP = 10240
_E = 160000
_H = 128
_FH = 64
_G = 50
_NI = 3
_CUTOFF = 10.0
_T = 12

_NTILES = 32
_CE = 4000
_BE = 6400
_BN = 2048
_BN0 = 1024


def _ssp(x):
    return (jnp.maximum(x, 0.0) + jnp.log1p(jnp.exp(-jnp.abs(x)))
            - math.log(2.0))


def _sc_dist_body(posT_hbm, eidx_hbm, d2_hbm, posb, srcb, dstb, outb):
    c = lax.axis_index("c")
    s = lax.axis_index("s")
    wid = s * 2 + c
    k = wid // 8
    ebase = (wid % 8) * (_E // 8)
    for r in range(3):
        pltpu.sync_copy(posT_hbm.at[k, r, :], posb.at[pl.ds(r * _N, _N)])

    def chunk(ci, carry):
        off = ebase + ci * _CE
        pltpu.sync_copy(eidx_hbm.at[k, 0, pl.ds(off, _CE)], srcb)
        pltpu.sync_copy(eidx_hbm.at[k, 1, pl.ds(off, _CE)], dstb)

        def inner(j, carry2):
            sv = srcb[pl.ds(j * 16, 16)]
            dv = dstb[pl.ds(j * 16, 16)]
            dx = (plsc.load_gather(posb, [sv])
                  - plsc.load_gather(posb, [dv]))
            dy = (plsc.load_gather(posb, [sv + _N])
                  - plsc.load_gather(posb, [dv + _N]))
            dz = (plsc.load_gather(posb, [sv + 2 * _N])
                  - plsc.load_gather(posb, [dv + 2 * _N]))
            outb[pl.ds(j * 16, 16)] = dx * dx + dy * dy + dz * dz
            return carry2

        lax.fori_loop(0, _CE // 16, inner, 0)
        pltpu.sync_copy(outb, d2_hbm.at[k, pl.ds(off, _CE)])
        return carry

    lax.fori_loop(0, (_E // 8) // _CE, chunk, 0)


def _sc_dist(posT, eidx):
    mesh = plsc.VectorSubcoreMesh(core_axis_name="c", subcore_axis_name="s")
    return pl.kernel(
        _sc_dist_body,
        out_type=jax.ShapeDtypeStruct((_K, _E), jnp.float32),
        mesh=mesh,
        scratch_types=[
            pltpu.VMEM((3 * _N,), jnp.float32),
            pltpu.VMEM((_CE,), jnp.int32),
            pltpu.VMEM((_CE,), jnp.int32),
            pltpu.VMEM((_CE,), jnp.float32),
        ],
    )(posT, eidx)


def _sc_scatter_body(x1T_hbm, wfT_hbm, eidx_hbm, aggT_hbm,
                     x1b, aggb, srcb, dstb, wf0b, wf1b):
    c = lax.axis_index("c")
    s = lax.axis_index("s")
    wid = s * 2 + c
    f0 = wid * 2
    zero16 = jnp.zeros((16,), jnp.float32)

    for k in range(_K):
        def zbody(j, carry):
            aggb[pl.ds(j * 16, 16)] = zero16
            return carry

        lax.fori_loop(0, (2 * _NP) // 16, zbody, 0)
        pltpu.sync_copy(x1T_hbm.at[k, f0, :], x1b.at[pl.ds(0, _NP)])
        pltpu.sync_copy(x1T_hbm.at[k, f0 + 1, :], x1b.at[pl.ds(_NP, _NP)])

        def chunk(ci, carry):
            off = ci * _CE
            pltpu.sync_copy(eidx_hbm.at[k, 0, pl.ds(off, _CE)], srcb)
            pltpu.sync_copy(eidx_hbm.at[k, 1, pl.ds(off, _CE)], dstb)
            pltpu.sync_copy(wfT_hbm.at[k, f0, pl.ds(off, _CE)], wf0b)
            pltpu.sync_copy(wfT_hbm.at[k, f0 + 1, pl.ds(off, _CE)], wf1b)

            def inner(j, carry2):
                sv = srcb[pl.ds(j * 16, 16)]
                dv = dstb[pl.ds(j * 16, 16)]
                g0 = plsc.load_gather(x1b, [sv])
                g1 = plsc.load_gather(x1b, [sv + _NP])
                m0 = g0 * wf0b[pl.ds(j * 16, 16)]
                m1 = g1 * wf1b[pl.ds(j * 16, 16)]
                plsc.addupdate_scatter(aggb, [dv], m0)
                plsc.addupdate_scatter(aggb, [dv + _NP], m1)
                return carry2

            lax.fori_loop(0, _CE // 16, inner, 0)
            return carry

        lax.fori_loop(0, _E // _CE, chunk, 0)
        pltpu.sync_copy(aggb.at[pl.ds(0, _NP)], aggT_hbm.at[k, f0, :])
        pltpu.sync_copy(aggb.at[pl.ds(_NP, _NP)], aggT_hbm.at[k, f0 + 1, :])


def _sc_scatter(x1T, wfT_i, eidx):
    mesh = plsc.VectorSubcoreMesh(core_axis_name="c", subcore_axis_name="s")
    return pl.kernel(
        _sc_scatter_body,
        out_type=jax.ShapeDtypeStruct((_K, _FH, _NP), jnp.float32),
        mesh=mesh,
        scratch_types=[
            pltpu.VMEM((2 * _NP,), jnp.float32),
            pltpu.VMEM((2 * _NP,), jnp.float32),
            pltpu.VMEM((_CE,), jnp.int32),
            pltpu.VMEM((_CE,), jnp.int32),
            pltpu.VMEM((_CE,), jnp.float32),
            pltpu.VMEM((_CE,), jnp.float32),
        ],
    )(x1T, wfT_i, eidx)


def _wf_body(d2_ref, w1T_ref, b1_ref, w2T_ref, b2_ref, out_ref):
    d2 = d2_ref[0, 0, 0, :]
    dist = jnp.sqrt(d2)
    delta = _CUTOFF / (_G - 1)
    coeff = -0.5 / (delta * delta)
    offs = lax.broadcasted_iota(jnp.float32, (_G, 1), 0) * delta
    diff = dist[None, :] - offs
    eaT = jnp.exp(coeff * (diff * diff))
    a = _ssp(jnp.dot(w1T_ref[0], eaT,
                     preferred_element_type=jnp.float32) + b1_ref[0])
    wf = (jnp.dot(w2T_ref[0], a, preferred_element_type=jnp.float32)
          + b2_ref[0])
    cc = 0.5 * (jnp.cos(dist * (math.pi / _CUTOFF)) + 1.0)
    out_ref[0, 0] = wf * cc[None, :]


def _tc_wf(d2, w1T, b1c, w2T, b2c):
    d2r = d2.reshape(_K, _E // _BE, 1, _BE)
    return pl.pallas_call(
        _wf_body,
        grid=(_NI, _K, _E // _BE),
        in_specs=[
            pl.BlockSpec((1, 1, 1, _BE), lambda i, k, e: (k, e, 0, 0)),
            pl.BlockSpec((1, _FH, _G), lambda i, k, e: (i, 0, 0)),
            pl.BlockSpec((1, _FH, 1), lambda i, k, e: (i, 0, 0)),
            pl.BlockSpec((1, _FH, _FH), lambda i, k, e: (i, 0, 0)),
            pl.BlockSpec((1, _FH, 1), lambda i, k, e: (i, 0, 0)),
        ],
        out_specs=pl.BlockSpec((1, 1, _FH, _BE), lambda i, k, e: (i, k, 0, e)),
        out_shape=jax.ShapeDtypeStruct((_NI, _K, _FH, _E), jnp.float32),
    )(d2r, w1T, b1c, w2T, b2c)


def _embed_body(zf_ref, embT_ref, out_ref):
    zrow = zf_ref[0, 0, :]
    ids = lax.broadcasted_iota(jnp.float32, (100, 1), 0)
    oh = (zrow[None, :] == ids).astype(jnp.float32)
    out_ref[...] = jnp.dot(embT_ref[...], oh,
                           preferred_element_type=jnp.float32)


def _tc_embed(zf3, embT):
    return pl.pallas_call(
        _embed_body,
        grid=(_NP // _BN0,),
        in_specs=[
            pl.BlockSpec((1, 1, _BN0), lambda b: (b, 0, 0)),
            pl.BlockSpec((_H, 100), lambda b: (0, 0)),
        ],
        out_specs=pl.BlockSpec((_H, _BN0), lambda b: (0, b)),
        out_shape=jax.ShapeDtypeStruct((_H, _NP), jnp.float32),
    )(zf3, embT)


def _x1_body(hT_ref, w_ref, out_ref):
    out_ref[0] = jnp.dot(w_ref[...], hT_ref[0],
                         preferred_element_type=jnp.float32)


def _tc_x1(hT, l1T_i):
    return pl.pallas_call(
        _x1_body,
        grid=(_K, _NP // _BN),
        in_specs=[
            pl.BlockSpec((1, _H, _BN), lambda k, n: (k, 0, n)),
            pl.BlockSpec((_FH, _H), lambda k, n: (0, 0)),
        ],
        out_specs=pl.BlockSpec((1, _FH, _BN), lambda k, n: (k, 0, n)),
        out_shape=jax.ShapeDtypeStruct((_K, _FH, _NP), jnp.float32),
    )(hT, l1T_i)


def _upd_body(aggT_ref, hT_ref, w2T_ref, b2_ref, wT_ref, b_ref, out_ref):
    a = _ssp(jnp.dot(w2T_ref[...], aggT_ref[0],
                     preferred_element_type=jnp.float32) + b2_ref[...])
    x2 = jnp.dot(wT_ref[...], a,
                 preferred_element_type=jnp.float32) + b_ref[...]
    out_ref[0] = hT_ref[0] + x2


def _tc_update(aggT, hT, l2T_i, b2c_i, lT_i, bc_i):
    return pl.pallas_call(
        _upd_body,
        grid=(_K, _NP // _BN),
        in_specs=[
            pl.BlockSpec((1, _FH, _BN), lambda k, n: (k, 0, n)),
            pl.BlockSpec((1, _H, _BN), lambda k, n: (k, 0, n)),
            pl.BlockSpec((_H, _FH), lambda k, n: (0, 0)),
            pl.BlockSpec((_H, 1), lambda k, n: (0, 0)),
            pl.BlockSpec((_H, _H), lambda k, n: (0, 0)),
            pl.BlockSpec((_H, 1), lambda k, n: (0, 0)),
        ],
        out_specs=pl.BlockSpec((1, _H, _BN), lambda k, n: (k, 0, n)),
        out_shape=jax.ShapeDtypeStruct((_K, _H, _NP), jnp.float32),
    )(aggT, hT, l2T_i, b2c_i, lT_i, bc_i)


def _mean_body(hT_ref, out_ref):
    x = hT_ref[0]
    msk = lax.broadcasted_iota(jnp.int32, (1, _NP), 1) < _N
    out_ref[0, 0] = jnp.sum(jnp.where(msk, x, 0.0), axis=1) * (1.0 / _N)


def _tc_mean(hT):
    return pl.pallas_call(
        _mean_body,
        grid=(_K,),
        in_specs=[pl.BlockSpec((1, _H, _NP), lambda k: (k, 0, 0))],
        out_specs=pl.BlockSpec((1, 1, _H), lambda k: (k, 0, 0)),
        out_shape=jax.ShapeDtypeStruct((_K, 1, _H), jnp.float32),
    )(hT)


def _head_body(r_ref, w1_ref, b1_ref, w2T_ref, b2_ref,
               cw1_ref, cb1_ref, cw2_ref, cb2_ref, out_ref):
    r = r_ref[...]
    t = jnp.tanh(jnp.dot(r, w1_ref[...],
                         preferred_element_type=jnp.float32) + b1_ref[...])
    sc = jnp.sum(t * w2T_ref[...], axis=1, keepdims=True) + b2_ref[...]
    m = jnp.max(sc, axis=0, keepdims=True)
    e = jnp.exp(sc - m)
    w = e / jnp.sum(e, axis=0, keepdims=True)
    fused = jnp.sum(w * r, axis=0, keepdims=True)
    hid = jnp.maximum(
        jnp.dot(fused, cw1_ref[...],
                preferred_element_type=jnp.float32) + cb1_ref[...], 0.0)
    out_ref[...] = jnp.dot(hid, cw2_ref[...],
                           preferred_element_type=jnp.float32) + cb2_ref[...]


def _tc_head(reprs, attn_w1, attn_b1r, attn_w2T, attn_b2r,
             cls_w1, cls_b1r, cls_w2, cls_b2r):
    return pl.pallas_call(
        _head_body,
        out_shape=jax.ShapeDtypeStruct((1, _T), jnp.float32),
    )(reprs, attn_w1, attn_b1r, attn_w2T, attn_b2r,
      cls_w1, cls_b1r, cls_w2, cls_b2r)


def kernel(z, pos, edge_index, emb, mlp_w1, mlp_b1, mlp_w2, mlp_b2,
           lin1_w, lin2_w, lin2_b, lin_w, lin_b,
           attn_w1, attn_b1, attn_w2, attn_b2,
           cls_w1, cls_b1, cls_w2, cls_b2):
    posT = jnp.transpose(pos, (0, 2, 1))
    eidx = edge_index.astype(jnp.int32)

    d2 = _sc_dist(posT, eidx)

    w1T = jnp.transpose(mlp_w1, (0, 2, 1))
    w2T = jnp.transpose(mlp_w2, (0, 2, 1))
    wfT = _tc_wf(d2, w1T, mlp_b1[:, :, None], w2T, mlp_b2[:, :, None])

    zf3 = jnp.pad(z.astype(jnp.float32),
                  (0, _NP - _N)).reshape(_NP // _BN0, 1, _BN0)
    hT0 = _tc_embed(zf3, emb.T)
    hT = jnp.broadcast_to(hT0[None], (_K, _H, _NP))

    for i in range(_NI):
        x1T = _tc_x1(hT, jnp.transpose(lin1_w[i]))
        aggT = _sc_scatter(x1T, wfT[i], eidx)
        hT = _tc_update(aggT, hT,
                        jnp.transpose(lin2_w[i]), lin2_b[i][:, None],
                        jnp.transpose(lin_w[i]), lin_b[i][:, None])

    reprs = _tc_mean(hT).reshape(_K, _H)
    out = _tc_head(reprs, attn_w1, attn_b1.reshape(1, _FH),
                   jnp.transpose(attn_w2), attn_b2.reshape(1, 1),
                   cls_w1, cls_b1.reshape(1, _H),
                   cls_w2, cls_b2.reshape(1, _T))
    return out.reshape(_T)


# speedup vs baseline: 2.1883x; 2.1883x over previous
import functools
import math

import jax
import jax.numpy as jnp
from jax import lax
from jax.experimental import pallas as pl
from jax.experimental.pallas import tpu as pltpu
from jax.experimental.pallas import tpu_sc as plsc

_K = 4
_N = 10000
_NP = 10240
_E = 160000
_H = 128
_FH = 64
_G = 50
_NI = 3
_CUTOFF = 10.0
_T = 12

_NTILES = 32
_CE = 4000
_BE = 6400
_BN = 2048
_BN0 = 1024


def _ssp(x):
    return (jnp.maximum(x, 0.0) + jnp.log1p(jnp.exp(-jnp.abs(x)))
            - math.log(2.0))


def _sc_dist_body(posT_hbm, eidx_hbm, d2_hbm, posb, srcb, dstb, outb):
    c = lax.axis_index("c")
    s = lax.axis_index("s")
    wid = s * 2 + c
    k = wid // 8
    ebase = (wid % 8) * (_E // 8)
    for r in range(3):
        pltpu.sync_copy(posT_hbm.at[pl.ds((k * 3 + r) * _NP, _NP)],
                        posb.at[pl.ds(r * _NP, _NP)])

    def chunk(ci, carry):
        off = ebase + ci * _CE
        pltpu.sync_copy(eidx_hbm.at[pl.ds(k * 2 * _E + off, _CE)], srcb)
        pltpu.sync_copy(eidx_hbm.at[pl.ds((k * 2 + 1) * _E + off, _CE)], dstb)

        def inner(j, carry2):
            sv = srcb[pl.ds(j * 16, 16)]
            dv = dstb[pl.ds(j * 16, 16)]
            dx = (plsc.load_gather(posb, [sv])
                  - plsc.load_gather(posb, [dv]))
            dy = (plsc.load_gather(posb, [sv + _NP])
                  - plsc.load_gather(posb, [dv + _NP]))
            dz = (plsc.load_gather(posb, [sv + 2 * _NP])
                  - plsc.load_gather(posb, [dv + 2 * _NP]))
            outb[pl.ds(j * 16, 16)] = dx * dx + dy * dy + dz * dz
            return carry2

        lax.fori_loop(0, _CE // 16, inner, 0)
        pltpu.sync_copy(outb, d2_hbm.at[pl.ds(k * _E + off, _CE)])
        return carry

    lax.fori_loop(0, (_E // 8) // _CE, chunk, 0)


def _sc_dist(posT, eidx):
    mesh = plsc.VectorSubcoreMesh(core_axis_name="c", subcore_axis_name="s")
    return pl.kernel(
        _sc_dist_body,
        out_type=jax.ShapeDtypeStruct((_K * _E,), jnp.float32),
        mesh=mesh,
        compiler_params=pltpu.CompilerParams(needs_layout_passes=False),
        scratch_types=[
            pltpu.VMEM((3 * _NP,), jnp.float32),
            pltpu.VMEM((_CE,), jnp.int32),
            pltpu.VMEM((_CE,), jnp.int32),
            pltpu.VMEM((_CE,), jnp.float32),
        ],
    )(posT, eidx)


def _sc_scatter_body(x1T_hbm, wfT_hbm, eidx_hbm, aggT_hbm,
                     x1b, aggb, srcb, dstb, wf0b, wf1b):
    c = lax.axis_index("c")
    s = lax.axis_index("s")
    wid = s * 2 + c
    f0 = wid * 2
    zero16 = jnp.zeros((16,), jnp.float32)

    for k in range(_K):
        def zbody(j, carry):
            aggb[pl.ds(j * 16, 16)] = zero16
            return carry

        lax.fori_loop(0, (2 * _NP) // 16, zbody, 0)
        nb = (k * _FH + f0) * _NP
        pltpu.sync_copy(x1T_hbm.at[pl.ds(nb, _NP)], x1b.at[pl.ds(0, _NP)])
        pltpu.sync_copy(x1T_hbm.at[pl.ds(nb + _NP, _NP)],
                        x1b.at[pl.ds(_NP, _NP)])

        def chunk(ci, carry):
            off = ci * _CE
            eb = (k * _FH + f0) * _E + off
            pltpu.sync_copy(eidx_hbm.at[pl.ds(k * 2 * _E + off, _CE)], srcb)
            pltpu.sync_copy(eidx_hbm.at[pl.ds((k * 2 + 1) * _E + off, _CE)],
                            dstb)
            pltpu.sync_copy(wfT_hbm.at[pl.ds(eb, _CE)], wf0b)
            pltpu.sync_copy(wfT_hbm.at[pl.ds(eb + _E, _CE)], wf1b)

            def inner(j, carry2):
                sv = srcb[pl.ds(j * 16, 16)]
                dv = dstb[pl.ds(j * 16, 16)]
                g0 = plsc.load_gather(x1b, [sv])
                g1 = plsc.load_gather(x1b, [sv + _NP])
                m0 = g0 * wf0b[pl.ds(j * 16, 16)]
                m1 = g1 * wf1b[pl.ds(j * 16, 16)]
                plsc.addupdate_scatter(aggb, [dv], m0)
                plsc.addupdate_scatter(aggb, [dv + _NP], m1)
                return carry2

            lax.fori_loop(0, _CE // 16, inner, 0)
            return carry

        lax.fori_loop(0, _E // _CE, chunk, 0)
        nb2 = (k * _FH + f0) * _NP
        pltpu.sync_copy(aggb.at[pl.ds(0, _NP)], aggT_hbm.at[pl.ds(nb2, _NP)])
        pltpu.sync_copy(aggb.at[pl.ds(_NP, _NP)],
                        aggT_hbm.at[pl.ds(nb2 + _NP, _NP)])


def _sc_scatter(x1T, wfT_i, eidx):
    mesh = plsc.VectorSubcoreMesh(core_axis_name="c", subcore_axis_name="s")
    return pl.kernel(
        _sc_scatter_body,
        out_type=jax.ShapeDtypeStruct((_K * _FH * _NP,), jnp.float32),
        mesh=mesh,
        compiler_params=pltpu.CompilerParams(needs_layout_passes=False),
        scratch_types=[
            pltpu.VMEM((2 * _NP,), jnp.float32),
            pltpu.VMEM((2 * _NP,), jnp.float32),
            pltpu.VMEM((_CE,), jnp.int32),
            pltpu.VMEM((_CE,), jnp.int32),
            pltpu.VMEM((_CE,), jnp.float32),
            pltpu.VMEM((_CE,), jnp.float32),
        ],
    )(x1T, wfT_i, eidx)


def _wf_body(d2_ref, w1T_ref, b1_ref, w2T_ref, b2_ref, out_ref):
    d2 = d2_ref[0, 0, 0, :]
    dist = jnp.sqrt(d2)
    delta = _CUTOFF / (_G - 1)
    coeff = -0.5 / (delta * delta)
    offs = lax.broadcasted_iota(jnp.int32, (_G, 1), 0).astype(jnp.float32) * delta
    diff = dist[None, :] - offs
    eaT = jnp.exp(coeff * (diff * diff))
    a = _ssp(jnp.dot(w1T_ref[0], eaT,
                     preferred_element_type=jnp.float32) + b1_ref[0])
    wf = (jnp.dot(w2T_ref[0], a, preferred_element_type=jnp.float32)
          + b2_ref[0])
    cc = 0.5 * (jnp.cos(dist * (math.pi / _CUTOFF)) + 1.0)
    out_ref[0, 0] = wf * cc[None, :]


def _tc_wf(d2, w1T, b1c, w2T, b2c):
    d2r = d2.reshape(_K, _E // _BE, 1, _BE)
    return pl.pallas_call(
        _wf_body,
        grid=(_NI, _K, _E // _BE),
        in_specs=[
            pl.BlockSpec((1, 1, 1, _BE), lambda i, k, e: (k, e, 0, 0)),
            pl.BlockSpec((1, _FH, _G), lambda i, k, e: (i, 0, 0)),
            pl.BlockSpec((1, _FH, 1), lambda i, k, e: (i, 0, 0)),
            pl.BlockSpec((1, _FH, _FH), lambda i, k, e: (i, 0, 0)),
            pl.BlockSpec((1, _FH, 1), lambda i, k, e: (i, 0, 0)),
        ],
        out_specs=pl.BlockSpec((1, 1, _FH, _BE), lambda i, k, e: (i, k, 0, e)),
        out_shape=jax.ShapeDtypeStruct((_NI, _K, _FH, _E), jnp.float32),
    )(d2r, w1T, b1c, w2T, b2c)


def _embed_body(zf_ref, embT_ref, out_ref):
    zrow = zf_ref[0, 0, :]
    ids = lax.broadcasted_iota(jnp.int32, (100, 1), 0).astype(jnp.float32)
    oh = (zrow[None, :] == ids).astype(jnp.float32)
    out_ref[...] = jnp.dot(embT_ref[...], oh,
                           preferred_element_type=jnp.float32)


def _tc_embed(zf3, embT):
    return pl.pallas_call(
        _embed_body,
        grid=(_NP // _BN0,),
        in_specs=[
            pl.BlockSpec((1, 1, _BN0), lambda b: (b, 0, 0)),
            pl.BlockSpec((_H, 100), lambda b: (0, 0)),
        ],
        out_specs=pl.BlockSpec((_H, _BN0), lambda b: (0, b)),
        out_shape=jax.ShapeDtypeStruct((_H, _NP), jnp.float32),
    )(zf3, embT)


def _x1_body(hT_ref, w_ref, out_ref):
    out_ref[0] = jnp.dot(w_ref[...], hT_ref[0],
                         preferred_element_type=jnp.float32)


def _tc_x1(hT, l1T_i):
    return pl.pallas_call(
        _x1_body,
        grid=(_K, _NP // _BN),
        in_specs=[
            pl.BlockSpec((1, _H, _BN), lambda k, n: (k, 0, n)),
            pl.BlockSpec((_FH, _H), lambda k, n: (0, 0)),
        ],
        out_specs=pl.BlockSpec((1, _FH, _BN), lambda k, n: (k, 0, n)),
        out_shape=jax.ShapeDtypeStruct((_K, _FH, _NP), jnp.float32),
    )(hT, l1T_i)


def _upd_body(aggT_ref, hT_ref, w2T_ref, b2_ref, wT_ref, b_ref, out_ref):
    a = _ssp(jnp.dot(w2T_ref[...], aggT_ref[0],
                     preferred_element_type=jnp.float32) + b2_ref[...])
    x2 = jnp.dot(wT_ref[...], a,
                 preferred_element_type=jnp.float32) + b_ref[...]
    out_ref[0] = hT_ref[0] + x2


def _tc_update(aggT, hT, l2T_i, b2c_i, lT_i, bc_i):
    return pl.pallas_call(
        _upd_body,
        grid=(_K, _NP // _BN),
        in_specs=[
            pl.BlockSpec((1, _FH, _BN), lambda k, n: (k, 0, n)),
            pl.BlockSpec((1, _H, _BN), lambda k, n: (k, 0, n)),
            pl.BlockSpec((_H, _FH), lambda k, n: (0, 0)),
            pl.BlockSpec((_H, 1), lambda k, n: (0, 0)),
            pl.BlockSpec((_H, _H), lambda k, n: (0, 0)),
            pl.BlockSpec((_H, 1), lambda k, n: (0, 0)),
        ],
        out_specs=pl.BlockSpec((1, _H, _BN), lambda k, n: (k, 0, n)),
        out_shape=jax.ShapeDtypeStruct((_K, _H, _NP), jnp.float32),
    )(aggT, hT, l2T_i, b2c_i, lT_i, bc_i)


def _mean_body(hT_ref, out_ref):
    x = hT_ref[0]
    msk = lax.broadcasted_iota(jnp.int32, (1, _NP), 1) < _N
    out_ref[0, 0] = jnp.sum(jnp.where(msk, x, 0.0), axis=1) * (1.0 / _N)


def _tc_mean(hT):
    return pl.pallas_call(
        _mean_body,
        grid=(_K,),
        in_specs=[pl.BlockSpec((1, _H, _NP), lambda k: (k, 0, 0))],
        out_specs=pl.BlockSpec((1, 1, _H), lambda k: (k, 0, 0)),
        out_shape=jax.ShapeDtypeStruct((_K, 1, _H), jnp.float32),
    )(hT)


def _head_body(r_ref, w1_ref, b1_ref, w2T_ref, b2_ref,
               cw1_ref, cb1_ref, cw2_ref, cb2_ref, out_ref):
    r = r_ref[...]
    t = jnp.tanh(jnp.dot(r, w1_ref[...],
                         preferred_element_type=jnp.float32) + b1_ref[...])
    sc = jnp.sum(t * w2T_ref[...], axis=1, keepdims=True) + b2_ref[...]
    m = jnp.max(sc, axis=0, keepdims=True)
    e = jnp.exp(sc - m)
    w = e / jnp.sum(e, axis=0, keepdims=True)
    fused = jnp.sum(w * r, axis=0, keepdims=True)
    hid = jnp.maximum(
        jnp.dot(fused, cw1_ref[...],
                preferred_element_type=jnp.float32) + cb1_ref[...], 0.0)
    out_ref[...] = jnp.dot(hid, cw2_ref[...],
                           preferred_element_type=jnp.float32) + cb2_ref[...]


def _tc_head(reprs, attn_w1, attn_b1r, attn_w2T, attn_b2r,
             cls_w1, cls_b1r, cls_w2, cls_b2r):
    return pl.pallas_call(
        _head_body,
        out_shape=jax.ShapeDtypeStruct((1, _T), jnp.float32),
    )(reprs, attn_w1, attn_b1r, attn_w2T, attn_b2r,
      cls_w1, cls_b1r, cls_w2, cls_b2r)


def kernel(z, pos, edge_index, emb, mlp_w1, mlp_b1, mlp_w2, mlp_b2,
           lin1_w, lin2_w, lin2_b, lin_w, lin_b,
           attn_w1, attn_b1, attn_w2, attn_b2,
           cls_w1, cls_b1, cls_w2, cls_b2):
    posT = jnp.pad(jnp.transpose(pos, (0, 2, 1)),
                   ((0, 0), (0, 0), (0, _NP - _N))).reshape(-1)
    eidx = edge_index.astype(jnp.int32).reshape(-1)

    d2 = _sc_dist(posT, eidx).reshape(_K, _E)

    w1T = jnp.transpose(mlp_w1, (0, 2, 1))
    w2T = jnp.transpose(mlp_w2, (0, 2, 1))
    wfT = _tc_wf(d2, w1T, mlp_b1[:, :, None], w2T, mlp_b2[:, :, None])

    zf3 = jnp.pad(z.astype(jnp.float32),
                  (0, _NP - _N)).reshape(_NP // _BN0, 1, _BN0)
    hT0 = _tc_embed(zf3, emb.T)
    hT = jnp.broadcast_to(hT0[None], (_K, _H, _NP))

    for i in range(_NI):
        x1T = _tc_x1(hT, jnp.transpose(lin1_w[i]))
        aggT = _sc_scatter(x1T.reshape(-1), wfT[i].reshape(-1),
                           eidx).reshape(_K, _FH, _NP)
        hT = _tc_update(aggT, hT,
                        jnp.transpose(lin2_w[i]), lin2_b[i][:, None],
                        jnp.transpose(lin_w[i]), lin_b[i][:, None])

    reprs = _tc_mean(hT).reshape(_K, _H)
    out = _tc_head(reprs, attn_w1, attn_b1.reshape(1, _FH),
                   jnp.transpose(attn_w2), attn_b2.reshape(1, 1),
                   cls_w1, cls_b1.reshape(1, _H),
                   cls_w2, cls_b2.reshape(1, _T))
    return out.reshape(_T)


# speedup vs baseline: 2.6159x; 1.1954x over previous
import functools
import math

import jax
import jax.numpy as jnp
from jax import lax
from jax.experimental import pallas as pl
from jax.experimental.pallas import tpu as pltpu
from jax.experimental.pallas import tpu_sc as plsc

_K = 4
_N = 10000
_NP = 10240
_E = 160000
_H = 128
_FH = 64
_G = 50
_NI = 3
_CUTOFF = 10.0
_T = 12

_NTILES = 32
_CE = 4000
_BE = 6400
_BN = 2048
_BN0 = 1024


def _ssp(x):
    return (jnp.maximum(x, 0.0) + jnp.log1p(jnp.exp(-jnp.abs(x)))
            - math.log(2.0))


def _sc_dist_body(posT_hbm, eidx_hbm, d2_hbm, posb, srcb, dstb, outb):
    c = lax.axis_index("c")
    s = lax.axis_index("s")
    wid = s * 2 + c
    k = wid // 8
    ebase = (wid % 8) * (_E // 8)
    for r in range(3):
        pltpu.sync_copy(posT_hbm.at[pl.ds((k * 3 + r) * _NP, _NP)],
                        posb.at[pl.ds(r * _NP, _NP)])

    def chunk(ci, carry):
        off = ebase + ci * _CE
        pltpu.sync_copy(eidx_hbm.at[pl.ds(k * 2 * _E + off, _CE)], srcb)
        pltpu.sync_copy(eidx_hbm.at[pl.ds((k * 2 + 1) * _E + off, _CE)], dstb)

        @plsc.parallel_loop(0, _CE // 16, unroll=4)
        def inner(j):
            sv = srcb[pl.ds(j * 16, 16)]
            dv = dstb[pl.ds(j * 16, 16)]
            dx = (plsc.load_gather(posb, [sv])
                  - plsc.load_gather(posb, [dv]))
            dy = (plsc.load_gather(posb, [sv + _NP])
                  - plsc.load_gather(posb, [dv + _NP]))
            dz = (plsc.load_gather(posb, [sv + 2 * _NP])
                  - plsc.load_gather(posb, [dv + 2 * _NP]))
            outb[pl.ds(j * 16, 16)] = dx * dx + dy * dy + dz * dz
        pltpu.sync_copy(outb, d2_hbm.at[pl.ds(k * _E + off, _CE)])
        return carry

    lax.fori_loop(0, (_E // 8) // _CE, chunk, 0)


def _sc_dist(posT, eidx):
    mesh = plsc.VectorSubcoreMesh(core_axis_name="c", subcore_axis_name="s")
    return pl.kernel(
        _sc_dist_body,
        out_type=jax.ShapeDtypeStruct((_K * _E,), jnp.float32),
        mesh=mesh,
        compiler_params=pltpu.CompilerParams(needs_layout_passes=False),
        scratch_types=[
            pltpu.VMEM((3 * _NP,), jnp.float32),
            pltpu.VMEM((_CE,), jnp.int32),
            pltpu.VMEM((_CE,), jnp.int32),
            pltpu.VMEM((_CE,), jnp.float32),
        ],
    )(posT, eidx)


def _sc_scatter_body(x1T_hbm, wfT_hbm, eidx_hbm, aggT_hbm,
                     x1b, aggb, srcb, dstb, wf0b, wf1b):
    c = lax.axis_index("c")
    s = lax.axis_index("s")
    wid = s * 2 + c
    f0 = wid * 2
    zero16 = jnp.zeros((16,), jnp.float32)

    for k in range(_K):
        def zbody(j, carry):
            aggb[pl.ds(j * 16, 16)] = zero16
            return carry

        lax.fori_loop(0, (2 * _NP) // 16, zbody, 0)
        nb = (k * _FH + f0) * _NP
        pltpu.sync_copy(x1T_hbm.at[pl.ds(nb, _NP)], x1b.at[pl.ds(0, _NP)])
        pltpu.sync_copy(x1T_hbm.at[pl.ds(nb + _NP, _NP)],
                        x1b.at[pl.ds(_NP, _NP)])

        def chunk(ci, carry):
            off = ci * _CE
            eb = (k * _FH + f0) * _E + off
            pltpu.sync_copy(eidx_hbm.at[pl.ds(k * 2 * _E + off, _CE)], srcb)
            pltpu.sync_copy(eidx_hbm.at[pl.ds((k * 2 + 1) * _E + off, _CE)],
                            dstb)
            pltpu.sync_copy(wfT_hbm.at[pl.ds(eb, _CE)], wf0b)
            pltpu.sync_copy(wfT_hbm.at[pl.ds(eb + _E, _CE)], wf1b)

            @plsc.parallel_loop(0, _CE // 16, unroll=4)
            def inner(j):
                sv = srcb[pl.ds(j * 16, 16)]
                dv = dstb[pl.ds(j * 16, 16)]
                g0 = plsc.load_gather(x1b, [sv])
                g1 = plsc.load_gather(x1b, [sv + _NP])
                m0 = g0 * wf0b[pl.ds(j * 16, 16)]
                m1 = g1 * wf1b[pl.ds(j * 16, 16)]
                plsc.addupdate_scatter(aggb, [dv], m0)
                plsc.addupdate_scatter(aggb, [dv + _NP], m1)

            return carry

        lax.fori_loop(0, _E // _CE, chunk, 0)
        nb2 = (k * _FH + f0) * _NP
        pltpu.sync_copy(aggb.at[pl.ds(0, _NP)], aggT_hbm.at[pl.ds(nb2, _NP)])
        pltpu.sync_copy(aggb.at[pl.ds(_NP, _NP)],
                        aggT_hbm.at[pl.ds(nb2 + _NP, _NP)])


def _sc_scatter(x1T, wfT_i, eidx):
    mesh = plsc.VectorSubcoreMesh(core_axis_name="c", subcore_axis_name="s")
    return pl.kernel(
        _sc_scatter_body,
        out_type=jax.ShapeDtypeStruct((_K * _FH * _NP,), jnp.float32),
        mesh=mesh,
        compiler_params=pltpu.CompilerParams(needs_layout_passes=False),
        scratch_types=[
            pltpu.VMEM((2 * _NP,), jnp.float32),
            pltpu.VMEM((2 * _NP,), jnp.float32),
            pltpu.VMEM((_CE,), jnp.int32),
            pltpu.VMEM((_CE,), jnp.int32),
            pltpu.VMEM((_CE,), jnp.float32),
            pltpu.VMEM((_CE,), jnp.float32),
        ],
    )(x1T, wfT_i, eidx)


def _wf_body(d2_ref, w1T_ref, b1_ref, w2T_ref, b2_ref, out_ref):
    d2 = d2_ref[0, 0, 0, :]
    dist = jnp.sqrt(d2)
    delta = _CUTOFF / (_G - 1)
    coeff = -0.5 / (delta * delta)
    offs = lax.broadcasted_iota(jnp.int32, (_G, 1), 0).astype(jnp.float32) * delta
    diff = dist[None, :] - offs
    eaT = jnp.exp(coeff * (diff * diff))
    a = _ssp(jnp.dot(w1T_ref[0], eaT,
                     preferred_element_type=jnp.float32) + b1_ref[0])
    wf = (jnp.dot(w2T_ref[0], a, preferred_element_type=jnp.float32)
          + b2_ref[0])
    cc = 0.5 * (jnp.cos(dist * (math.pi / _CUTOFF)) + 1.0)
    out_ref[0, 0] = wf * cc[None, :]


def _tc_wf(d2, w1T, b1c, w2T, b2c):
    d2r = d2.reshape(_K, _E // _BE, 1, _BE)
    return pl.pallas_call(
        _wf_body,
        grid=(_NI, _K, _E // _BE),
        in_specs=[
            pl.BlockSpec((1, 1, 1, _BE), lambda i, k, e: (k, e, 0, 0)),
            pl.BlockSpec((1, _FH, _G), lambda i, k, e: (i, 0, 0)),
            pl.BlockSpec((1, _FH, 1), lambda i, k, e: (i, 0, 0)),
            pl.BlockSpec((1, _FH, _FH), lambda i, k, e: (i, 0, 0)),
            pl.BlockSpec((1, _FH, 1), lambda i, k, e: (i, 0, 0)),
        ],
        out_specs=pl.BlockSpec((1, 1, _FH, _BE), lambda i, k, e: (i, k, 0, e)),
        out_shape=jax.ShapeDtypeStruct((_NI, _K, _FH, _E), jnp.float32),
    )(d2r, w1T, b1c, w2T, b2c)


def _embed_body(zf_ref, embT_ref, out_ref):
    zrow = zf_ref[0, 0, :]
    ids = lax.broadcasted_iota(jnp.int32, (100, 1), 0).astype(jnp.float32)
    oh = (zrow[None, :] == ids).astype(jnp.float32)
    out_ref[...] = jnp.dot(embT_ref[...], oh,
                           preferred_element_type=jnp.float32)


def _tc_embed(zf3, embT):
    return pl.pallas_call(
        _embed_body,
        grid=(_NP // _BN0,),
        in_specs=[
            pl.BlockSpec((1, 1, _BN0), lambda b: (b, 0, 0)),
            pl.BlockSpec((_H, 100), lambda b: (0, 0)),
        ],
        out_specs=pl.BlockSpec((_H, _BN0), lambda b: (0, b)),
        out_shape=jax.ShapeDtypeStruct((_H, _NP), jnp.float32),
    )(zf3, embT)


def _x1_body(hT_ref, w_ref, out_ref):
    out_ref[0] = jnp.dot(w_ref[...], hT_ref[0],
                         preferred_element_type=jnp.float32)


def _tc_x1(hT, l1T_i):
    return pl.pallas_call(
        _x1_body,
        grid=(_K, _NP // _BN),
        in_specs=[
            pl.BlockSpec((1, _H, _BN), lambda k, n: (k, 0, n)),
            pl.BlockSpec((_FH, _H), lambda k, n: (0, 0)),
        ],
        out_specs=pl.BlockSpec((1, _FH, _BN), lambda k, n: (k, 0, n)),
        out_shape=jax.ShapeDtypeStruct((_K, _FH, _NP), jnp.float32),
    )(hT, l1T_i)


def _upd_body(aggT_ref, hT_ref, w2T_ref, b2_ref, wT_ref, b_ref, out_ref):
    a = _ssp(jnp.dot(w2T_ref[...], aggT_ref[0],
                     preferred_element_type=jnp.float32) + b2_ref[...])
    x2 = jnp.dot(wT_ref[...], a,
                 preferred_element_type=jnp.float32) + b_ref[...]
    out_ref[0] = hT_ref[0] + x2


def _tc_update(aggT, hT, l2T_i, b2c_i, lT_i, bc_i):
    return pl.pallas_call(
        _upd_body,
        grid=(_K, _NP // _BN),
        in_specs=[
            pl.BlockSpec((1, _FH, _BN), lambda k, n: (k, 0, n)),
            pl.BlockSpec((1, _H, _BN), lambda k, n: (k, 0, n)),
            pl.BlockSpec((_H, _FH), lambda k, n: (0, 0)),
            pl.BlockSpec((_H, 1), lambda k, n: (0, 0)),
            pl.BlockSpec((_H, _H), lambda k, n: (0, 0)),
            pl.BlockSpec((_H, 1), lambda k, n: (0, 0)),
        ],
        out_specs=pl.BlockSpec((1, _H, _BN), lambda k, n: (k, 0, n)),
        out_shape=jax.ShapeDtypeStruct((_K, _H, _NP), jnp.float32),
    )(aggT, hT, l2T_i, b2c_i, lT_i, bc_i)


def _mean_body(hT_ref, out_ref):
    x = hT_ref[0]
    msk = lax.broadcasted_iota(jnp.int32, (1, _NP), 1) < _N
    out_ref[0, 0] = jnp.sum(jnp.where(msk, x, 0.0), axis=1) * (1.0 / _N)


def _tc_mean(hT):
    return pl.pallas_call(
        _mean_body,
        grid=(_K,),
        in_specs=[pl.BlockSpec((1, _H, _NP), lambda k: (k, 0, 0))],
        out_specs=pl.BlockSpec((1, 1, _H), lambda k: (k, 0, 0)),
        out_shape=jax.ShapeDtypeStruct((_K, 1, _H), jnp.float32),
    )(hT)


def _head_body(r_ref, w1_ref, b1_ref, w2T_ref, b2_ref,
               cw1_ref, cb1_ref, cw2_ref, cb2_ref, out_ref):
    r = r_ref[...]
    t = jnp.tanh(jnp.dot(r, w1_ref[...],
                         preferred_element_type=jnp.float32) + b1_ref[...])
    sc = jnp.sum(t * w2T_ref[...], axis=1, keepdims=True) + b2_ref[...]
    m = jnp.max(sc, axis=0, keepdims=True)
    e = jnp.exp(sc - m)
    w = e / jnp.sum(e, axis=0, keepdims=True)
    fused = jnp.sum(w * r, axis=0, keepdims=True)
    hid = jnp.maximum(
        jnp.dot(fused, cw1_ref[...],
                preferred_element_type=jnp.float32) + cb1_ref[...], 0.0)
    out_ref[...] = jnp.dot(hid, cw2_ref[...],
                           preferred_element_type=jnp.float32) + cb2_ref[...]


def _tc_head(reprs, attn_w1, attn_b1r, attn_w2T, attn_b2r,
             cls_w1, cls_b1r, cls_w2, cls_b2r):
    return pl.pallas_call(
        _head_body,
        out_shape=jax.ShapeDtypeStruct((1, _T), jnp.float32),
    )(reprs, attn_w1, attn_b1r, attn_w2T, attn_b2r,
      cls_w1, cls_b1r, cls_w2, cls_b2r)


def kernel(z, pos, edge_index, emb, mlp_w1, mlp_b1, mlp_w2, mlp_b2,
           lin1_w, lin2_w, lin2_b, lin_w, lin_b,
           attn_w1, attn_b1, attn_w2, attn_b2,
           cls_w1, cls_b1, cls_w2, cls_b2):
    posT = jnp.pad(jnp.transpose(pos, (0, 2, 1)),
                   ((0, 0), (0, 0), (0, _NP - _N))).reshape(-1)
    eidx = edge_index.astype(jnp.int32).reshape(-1)

    d2 = _sc_dist(posT, eidx).reshape(_K, _E)

    w1T = jnp.transpose(mlp_w1, (0, 2, 1))
    w2T = jnp.transpose(mlp_w2, (0, 2, 1))
    wfT = _tc_wf(d2, w1T, mlp_b1[:, :, None], w2T, mlp_b2[:, :, None])

    zf3 = jnp.pad(z.astype(jnp.float32),
                  (0, _NP - _N)).reshape(_NP // _BN0, 1, _BN0)
    hT0 = _tc_embed(zf3, emb.T)
    hT = jnp.broadcast_to(hT0[None], (_K, _H, _NP))

    for i in range(_NI):
        x1T = _tc_x1(hT, jnp.transpose(lin1_w[i]))
        aggT = _sc_scatter(x1T.reshape(-1), wfT[i].reshape(-1),
                           eidx).reshape(_K, _FH, _NP)
        hT = _tc_update(aggT, hT,
                        jnp.transpose(lin2_w[i]), lin2_b[i][:, None],
                        jnp.transpose(lin_w[i]), lin_b[i][:, None])

    reprs = _tc_mean(hT).reshape(_K, _H)
    out = _tc_head(reprs, attn_w1, attn_b1.reshape(1, _FH),
                   jnp.transpose(attn_w2), attn_b2.reshape(1, 1),
                   cls_w1, cls_b1.reshape(1, _H),
                   cls_w2, cls_b2.reshape(1, _T))
    return out.reshape(_T)


# speedup vs baseline: 2.6426x; 1.0102x over previous
import functools
import math

import jax
import jax.numpy as jnp
from jax import lax
from jax.experimental import pallas as pl
from jax.experimental.pallas import tpu as pltpu
from jax.experimental.pallas import tpu_sc as plsc

_K = 4
_N = 10000
_NP = 10240
_E = 160000
_H = 128
_FH = 64
_G = 50
_NI = 3
_CUTOFF = 10.0
_T = 12

_NTILES = 32
_CE = 4000
_BE = 6400
_BN = 2048
_BN0 = 1024


def _ssp(x):
    return (jnp.maximum(x, 0.0) + jnp.log1p(jnp.exp(-jnp.abs(x)))
            - math.log(2.0))


def _sc_dist_body(posT_hbm, eidx_hbm, d2_hbm, posb, srcb, dstb, outb):
    c = lax.axis_index("c")
    s = lax.axis_index("s")
    wid = s * 2 + c
    k = wid // 8
    ebase = (wid % 8) * (_E // 8)
    for r in range(3):
        pltpu.sync_copy(posT_hbm.at[pl.ds((k * 3 + r) * _NP, _NP)],
                        posb.at[pl.ds(r * _NP, _NP)])

    def chunk(ci, carry):
        off = ebase + ci * _CE
        pltpu.sync_copy(eidx_hbm.at[pl.ds(k * 2 * _E + off, _CE)], srcb)
        pltpu.sync_copy(eidx_hbm.at[pl.ds((k * 2 + 1) * _E + off, _CE)], dstb)

        @plsc.parallel_loop(0, _CE // 16, unroll=8)
        def inner(j):
            sv = srcb[pl.ds(j * 16, 16)]
            dv = dstb[pl.ds(j * 16, 16)]
            dx = (plsc.load_gather(posb, [sv])
                  - plsc.load_gather(posb, [dv]))
            dy = (plsc.load_gather(posb, [sv + _NP])
                  - plsc.load_gather(posb, [dv + _NP]))
            dz = (plsc.load_gather(posb, [sv + 2 * _NP])
                  - plsc.load_gather(posb, [dv + 2 * _NP]))
            outb[pl.ds(j * 16, 16)] = dx * dx + dy * dy + dz * dz
        pltpu.sync_copy(outb, d2_hbm.at[pl.ds(k * _E + off, _CE)])
        return carry

    lax.fori_loop(0, (_E // 8) // _CE, chunk, 0)


def _sc_dist(posT, eidx):
    mesh = plsc.VectorSubcoreMesh(core_axis_name="c", subcore_axis_name="s")
    return pl.kernel(
        _sc_dist_body,
        out_type=jax.ShapeDtypeStruct((_K * _E,), jnp.float32),
        mesh=mesh,
        compiler_params=pltpu.CompilerParams(needs_layout_passes=False),
        scratch_types=[
            pltpu.VMEM((3 * _NP,), jnp.float32),
            pltpu.VMEM((_CE,), jnp.int32),
            pltpu.VMEM((_CE,), jnp.int32),
            pltpu.VMEM((_CE,), jnp.float32),
        ],
    )(posT, eidx)


def _sc_scatter_body(x1T_hbm, wfT_hbm, eidx_hbm, aggT_hbm,
                     x1b, aggb, srcb, dstb, wf0b, wf1b):
    c = lax.axis_index("c")
    s = lax.axis_index("s")
    wid = s * 2 + c
    f0 = wid * 2
    zero16 = jnp.zeros((16,), jnp.float32)

    for k in range(_K):
        def zbody(j, carry):
            aggb[pl.ds(j * 16, 16)] = zero16
            return carry

        lax.fori_loop(0, (2 * _NP) // 16, zbody, 0)
        nb = (k * _FH + f0) * _NP
        pltpu.sync_copy(x1T_hbm.at[pl.ds(nb, _NP)], x1b.at[pl.ds(0, _NP)])
        pltpu.sync_copy(x1T_hbm.at[pl.ds(nb + _NP, _NP)],
                        x1b.at[pl.ds(_NP, _NP)])

        def chunk(ci, carry):
            off = ci * _CE
            eb = (k * _FH + f0) * _E + off
            pltpu.sync_copy(eidx_hbm.at[pl.ds(k * 2 * _E + off, _CE)], srcb)
            pltpu.sync_copy(eidx_hbm.at[pl.ds((k * 2 + 1) * _E + off, _CE)],
                            dstb)
            pltpu.sync_copy(wfT_hbm.at[pl.ds(eb, _CE)], wf0b)
            pltpu.sync_copy(wfT_hbm.at[pl.ds(eb + _E, _CE)], wf1b)

            @plsc.parallel_loop(0, _CE // 16, unroll=8)
            def inner(j):
                sv = srcb[pl.ds(j * 16, 16)]
                dv = dstb[pl.ds(j * 16, 16)]
                g0 = plsc.load_gather(x1b, [sv])
                g1 = plsc.load_gather(x1b, [sv + _NP])
                m0 = g0 * wf0b[pl.ds(j * 16, 16)]
                m1 = g1 * wf1b[pl.ds(j * 16, 16)]
                plsc.addupdate_scatter(aggb, [dv], m0)
                plsc.addupdate_scatter(aggb, [dv + _NP], m1)

            return carry

        lax.fori_loop(0, _E // _CE, chunk, 0)
        nb2 = (k * _FH + f0) * _NP
        pltpu.sync_copy(aggb.at[pl.ds(0, _NP)], aggT_hbm.at[pl.ds(nb2, _NP)])
        pltpu.sync_copy(aggb.at[pl.ds(_NP, _NP)],
                        aggT_hbm.at[pl.ds(nb2 + _NP, _NP)])


def _sc_scatter(x1T, wfT_i, eidx):
    mesh = plsc.VectorSubcoreMesh(core_axis_name="c", subcore_axis_name="s")
    return pl.kernel(
        _sc_scatter_body,
        out_type=jax.ShapeDtypeStruct((_K * _FH * _NP,), jnp.float32),
        mesh=mesh,
        compiler_params=pltpu.CompilerParams(needs_layout_passes=False),
        scratch_types=[
            pltpu.VMEM((2 * _NP,), jnp.float32),
            pltpu.VMEM((2 * _NP,), jnp.float32),
            pltpu.VMEM((_CE,), jnp.int32),
            pltpu.VMEM((_CE,), jnp.int32),
            pltpu.VMEM((_CE,), jnp.float32),
            pltpu.VMEM((_CE,), jnp.float32),
        ],
    )(x1T, wfT_i, eidx)


def _wf_body(d2_ref, w1T_ref, b1_ref, w2T_ref, b2_ref, out_ref):
    d2 = d2_ref[0, 0, 0, :]
    dist = jnp.sqrt(d2)
    delta = _CUTOFF / (_G - 1)
    coeff = -0.5 / (delta * delta)
    offs = lax.broadcasted_iota(jnp.int32, (_G, 1), 0).astype(jnp.float32) * delta
    diff = dist[None, :] - offs
    eaT = jnp.exp(coeff * (diff * diff))
    a = _ssp(jnp.dot(w1T_ref[0], eaT,
                     preferred_element_type=jnp.float32) + b1_ref[0])
    wf = (jnp.dot(w2T_ref[0], a, preferred_element_type=jnp.float32)
          + b2_ref[0])
    cc = 0.5 * (jnp.cos(dist * (math.pi / _CUTOFF)) + 1.0)
    out_ref[0, 0] = wf * cc[None, :]


def _tc_wf(d2, w1T, b1c, w2T, b2c):
    d2r = d2.reshape(_K, _E // _BE, 1, _BE)
    return pl.pallas_call(
        _wf_body,
        grid=(_NI, _K, _E // _BE),
        in_specs=[
            pl.BlockSpec((1, 1, 1, _BE), lambda i, k, e: (k, e, 0, 0)),
            pl.BlockSpec((1, _FH, _G), lambda i, k, e: (i, 0, 0)),
            pl.BlockSpec((1, _FH, 1), lambda i, k, e: (i, 0, 0)),
            pl.BlockSpec((1, _FH, _FH), lambda i, k, e: (i, 0, 0)),
            pl.BlockSpec((1, _FH, 1), lambda i, k, e: (i, 0, 0)),
        ],
        out_specs=pl.BlockSpec((1, 1, _FH, _BE), lambda i, k, e: (i, k, 0, e)),
        out_shape=jax.ShapeDtypeStruct((_NI, _K, _FH, _E), jnp.float32),
    )(d2r, w1T, b1c, w2T, b2c)


def _embed_body(zf_ref, embT_ref, out_ref):
    zrow = zf_ref[0, 0, :]
    ids = lax.broadcasted_iota(jnp.int32, (100, 1), 0).astype(jnp.float32)
    oh = (zrow[None, :] == ids).astype(jnp.float32)
    out_ref[...] = jnp.dot(embT_ref[...], oh,
                           preferred_element_type=jnp.float32)


def _tc_embed(zf3, embT):
    return pl.pallas_call(
        _embed_body,
        grid=(_NP // _BN0,),
        in_specs=[
            pl.BlockSpec((1, 1, _BN0), lambda b: (b, 0, 0)),
            pl.BlockSpec((_H, 100), lambda b: (0, 0)),
        ],
        out_specs=pl.BlockSpec((_H, _BN0), lambda b: (0, b)),
        out_shape=jax.ShapeDtypeStruct((_H, _NP), jnp.float32),
    )(zf3, embT)


def _x1_body(hT_ref, w_ref, out_ref):
    out_ref[0] = jnp.dot(w_ref[...], hT_ref[0],
                         preferred_element_type=jnp.float32)


def _tc_x1(hT, l1T_i):
    return pl.pallas_call(
        _x1_body,
        grid=(_K, _NP // _BN),
        in_specs=[
            pl.BlockSpec((1, _H, _BN), lambda k, n: (k, 0, n)),
            pl.BlockSpec((_FH, _H), lambda k, n: (0, 0)),
        ],
        out_specs=pl.BlockSpec((1, _FH, _BN), lambda k, n: (k, 0, n)),
        out_shape=jax.ShapeDtypeStruct((_K, _FH, _NP), jnp.float32),
    )(hT, l1T_i)


def _upd_body(aggT_ref, hT_ref, w2T_ref, b2_ref, wT_ref, b_ref, out_ref):
    a = _ssp(jnp.dot(w2T_ref[...], aggT_ref[0],
                     preferred_element_type=jnp.float32) + b2_ref[...])
    x2 = jnp.dot(wT_ref[...], a,
                 preferred_element_type=jnp.float32) + b_ref[...]
    out_ref[0] = hT_ref[0] + x2


def _tc_update(aggT, hT, l2T_i, b2c_i, lT_i, bc_i):
    return pl.pallas_call(
        _upd_body,
        grid=(_K, _NP // _BN),
        in_specs=[
            pl.BlockSpec((1, _FH, _BN), lambda k, n: (k, 0, n)),
            pl.BlockSpec((1, _H, _BN), lambda k, n: (k, 0, n)),
            pl.BlockSpec((_H, _FH), lambda k, n: (0, 0)),
            pl.BlockSpec((_H, 1), lambda k, n: (0, 0)),
            pl.BlockSpec((_H, _H), lambda k, n: (0, 0)),
            pl.BlockSpec((_H, 1), lambda k, n: (0, 0)),
        ],
        out_specs=pl.BlockSpec((1, _H, _BN), lambda k, n: (k, 0, n)),
        out_shape=jax.ShapeDtypeStruct((_K, _H, _NP), jnp.float32),
    )(aggT, hT, l2T_i, b2c_i, lT_i, bc_i)


def _mean_body(hT_ref, out_ref):
    x = hT_ref[0]
    msk = lax.broadcasted_iota(jnp.int32, (1, _NP), 1) < _N
    out_ref[0, 0] = jnp.sum(jnp.where(msk, x, 0.0), axis=1) * (1.0 / _N)


def _tc_mean(hT):
    return pl.pallas_call(
        _mean_body,
        grid=(_K,),
        in_specs=[pl.BlockSpec((1, _H, _NP), lambda k: (k, 0, 0))],
        out_specs=pl.BlockSpec((1, 1, _H), lambda k: (k, 0, 0)),
        out_shape=jax.ShapeDtypeStruct((_K, 1, _H), jnp.float32),
    )(hT)


def _head_body(r_ref, w1_ref, b1_ref, w2T_ref, b2_ref,
               cw1_ref, cb1_ref, cw2_ref, cb2_ref, out_ref):
    r = r_ref[...]
    t = jnp.tanh(jnp.dot(r, w1_ref[...],
                         preferred_element_type=jnp.float32) + b1_ref[...])
    sc = jnp.sum(t * w2T_ref[...], axis=1, keepdims=True) + b2_ref[...]
    m = jnp.max(sc, axis=0, keepdims=True)
    e = jnp.exp(sc - m)
    w = e / jnp.sum(e, axis=0, keepdims=True)
    fused = jnp.sum(w * r, axis=0, keepdims=True)
    hid = jnp.maximum(
        jnp.dot(fused, cw1_ref[...],
                preferred_element_type=jnp.float32) + cb1_ref[...], 0.0)
    out_ref[...] = jnp.dot(hid, cw2_ref[...],
                           preferred_element_type=jnp.float32) + cb2_ref[...]


def _tc_head(reprs, attn_w1, attn_b1r, attn_w2T, attn_b2r,
             cls_w1, cls_b1r, cls_w2, cls_b2r):
    return pl.pallas_call(
        _head_body,
        out_shape=jax.ShapeDtypeStruct((1, _T), jnp.float32),
    )(reprs, attn_w1, attn_b1r, attn_w2T, attn_b2r,
      cls_w1, cls_b1r, cls_w2, cls_b2r)


def kernel(z, pos, edge_index, emb, mlp_w1, mlp_b1, mlp_w2, mlp_b2,
           lin1_w, lin2_w, lin2_b, lin_w, lin_b,
           attn_w1, attn_b1, attn_w2, attn_b2,
           cls_w1, cls_b1, cls_w2, cls_b2):
    posT = jnp.pad(jnp.transpose(pos, (0, 2, 1)),
                   ((0, 0), (0, 0), (0, _NP - _N))).reshape(-1)
    eidx = edge_index.astype(jnp.int32).reshape(-1)

    d2 = _sc_dist(posT, eidx).reshape(_K, _E)

    w1T = jnp.transpose(mlp_w1, (0, 2, 1))
    w2T = jnp.transpose(mlp_w2, (0, 2, 1))
    wfT = _tc_wf(d2, w1T, mlp_b1[:, :, None], w2T, mlp_b2[:, :, None])

    zf3 = jnp.pad(z.astype(jnp.float32),
                  (0, _NP - _N)).reshape(_NP // _BN0, 1, _BN0)
    hT0 = _tc_embed(zf3, emb.T)
    hT = jnp.broadcast_to(hT0[None], (_K, _H, _NP))

    for i in range(_NI):
        x1T = _tc_x1(hT, jnp.transpose(lin1_w[i]))
        aggT = _sc_scatter(x1T.reshape(-1), wfT[i].reshape(-1),
                           eidx).reshape(_K, _FH, _NP)
        hT = _tc_update(aggT, hT,
                        jnp.transpose(lin2_w[i]), lin2_b[i][:, None],
                        jnp.transpose(lin_w[i]), lin_b[i][:, None])

    reprs = _tc_mean(hT).reshape(_K, _H)
    out = _tc_head(reprs, attn_w1, attn_b1.reshape(1, _FH),
                   jnp.transpose(attn_w2), attn_b2.reshape(1, 1),
                   cls_w1, cls_b1.reshape(1, _H),
                   cls_w2, cls_b2.reshape(1, _T))
    return out.reshape(_T)


# speedup vs baseline: 3.1979x; 1.2101x over previous
import functools
import math

import jax
import jax.numpy as jnp
from jax import lax
from jax.experimental import pallas as pl
from jax.experimental.pallas import tpu as pltpu
from jax.experimental.pallas import tpu_sc as plsc

_K = 4
_N = 10000
_NP = 10240
_E = 160000
_H = 128
_FH = 64
_G = 50
_NI = 3
_CUTOFF = 10.0
_T = 12

_NTILES = 32
_CE = 10000
_BE = 6400
_BN = 2048
_BN0 = 1024


def _ssp(x):
    return (jnp.maximum(x, 0.0) + jnp.log1p(jnp.exp(-jnp.abs(x)))
            - math.log(2.0))


def _sc_dist_body(posT_hbm, eidx_hbm, d2_hbm, posb, srcb, dstb, outb):
    c = lax.axis_index("c")
    s = lax.axis_index("s")
    wid = s * 2 + c
    k = wid // 8
    ebase = (wid % 8) * (_E // 8)
    for r in range(3):
        pltpu.sync_copy(posT_hbm.at[pl.ds((k * 3 + r) * _NP, _NP)],
                        posb.at[pl.ds(r * _NP, _NP)])

    def chunk(ci, carry):
        off = ebase + ci * _CE
        pltpu.sync_copy(eidx_hbm.at[pl.ds(k * 2 * _E + off, _CE)], srcb)
        pltpu.sync_copy(eidx_hbm.at[pl.ds((k * 2 + 1) * _E + off, _CE)], dstb)

        @plsc.parallel_loop(0, _CE // 16, unroll=8)
        def inner(j):
            sv = srcb[pl.ds(j * 16, 16)]
            dv = dstb[pl.ds(j * 16, 16)]
            dx = (plsc.load_gather(posb, [sv])
                  - plsc.load_gather(posb, [dv]))
            dy = (plsc.load_gather(posb, [sv + _NP])
                  - plsc.load_gather(posb, [dv + _NP]))
            dz = (plsc.load_gather(posb, [sv + 2 * _NP])
                  - plsc.load_gather(posb, [dv + 2 * _NP]))
            outb[pl.ds(j * 16, 16)] = dx * dx + dy * dy + dz * dz
        pltpu.sync_copy(outb, d2_hbm.at[pl.ds(k * _E + off, _CE)])
        return carry

    lax.fori_loop(0, (_E // 8) // _CE, chunk, 0)


def _sc_dist(posT, eidx):
    mesh = plsc.VectorSubcoreMesh(core_axis_name="c", subcore_axis_name="s")
    return pl.kernel(
        _sc_dist_body,
        out_type=jax.ShapeDtypeStruct((_K * _E,), jnp.float32),
        mesh=mesh,
        compiler_params=pltpu.CompilerParams(needs_layout_passes=False),
        scratch_types=[
            pltpu.VMEM((3 * _NP,), jnp.float32),
            pltpu.VMEM((_CE,), jnp.int32),
            pltpu.VMEM((_CE,), jnp.int32),
            pltpu.VMEM((_CE,), jnp.float32),
        ],
    )(posT, eidx)


def _sc_scatter_body(x1T_hbm, wfT_hbm, eidx_hbm, aggT_hbm,
                     x1b, aggb, srcb, dstb, wf0b, wf1b):
    c = lax.axis_index("c")
    s = lax.axis_index("s")
    wid = s * 2 + c
    f0 = wid * 2
    zero16 = jnp.zeros((16,), jnp.float32)

    for k in range(_K):
        def zbody(j, carry):
            aggb[pl.ds(j * 16, 16)] = zero16
            return carry

        lax.fori_loop(0, (2 * _NP) // 16, zbody, 0)
        nb = (k * _FH + f0) * _NP
        pltpu.sync_copy(x1T_hbm.at[pl.ds(nb, _NP)], x1b.at[pl.ds(0, _NP)])
        pltpu.sync_copy(x1T_hbm.at[pl.ds(nb + _NP, _NP)],
                        x1b.at[pl.ds(_NP, _NP)])

        def chunk(ci, carry):
            off = ci * _CE
            eb = (k * _FH + f0) * _E + off
            pltpu.sync_copy(eidx_hbm.at[pl.ds(k * 2 * _E + off, _CE)], srcb)
            pltpu.sync_copy(eidx_hbm.at[pl.ds((k * 2 + 1) * _E + off, _CE)],
                            dstb)
            pltpu.sync_copy(wfT_hbm.at[pl.ds(eb, _CE)], wf0b)
            pltpu.sync_copy(wfT_hbm.at[pl.ds(eb + _E, _CE)], wf1b)

            @plsc.parallel_loop(0, _CE // 16, unroll=8)
            def inner(j):
                sv = srcb[pl.ds(j * 16, 16)]
                dv = dstb[pl.ds(j * 16, 16)]
                g0 = plsc.load_gather(x1b, [sv])
                g1 = plsc.load_gather(x1b, [sv + _NP])
                m0 = g0 * wf0b[pl.ds(j * 16, 16)]
                m1 = g1 * wf1b[pl.ds(j * 16, 16)]
                plsc.addupdate_scatter(aggb, [dv], m0)
                plsc.addupdate_scatter(aggb, [dv + _NP], m1)

            return carry

        lax.fori_loop(0, _E // _CE, chunk, 0)
        nb2 = (k * _FH + f0) * _NP
        pltpu.sync_copy(aggb.at[pl.ds(0, _NP)], aggT_hbm.at[pl.ds(nb2, _NP)])
        pltpu.sync_copy(aggb.at[pl.ds(_NP, _NP)],
                        aggT_hbm.at[pl.ds(nb2 + _NP, _NP)])


def _sc_scatter(x1T, wfT_i, eidx):
    mesh = plsc.VectorSubcoreMesh(core_axis_name="c", subcore_axis_name="s")
    return pl.kernel(
        _sc_scatter_body,
        out_type=jax.ShapeDtypeStruct((_K * _FH * _NP,), jnp.float32),
        mesh=mesh,
        compiler_params=pltpu.CompilerParams(needs_layout_passes=False),
        scratch_types=[
            pltpu.VMEM((2 * _NP,), jnp.float32),
            pltpu.VMEM((2 * _NP,), jnp.float32),
            pltpu.VMEM((_CE,), jnp.int32),
            pltpu.VMEM((_CE,), jnp.int32),
            pltpu.VMEM((_CE,), jnp.float32),
            pltpu.VMEM((_CE,), jnp.float32),
        ],
    )(x1T, wfT_i, eidx)


def _wf_body(d2_ref, w1T_ref, b1_ref, w2T_ref, b2_ref, out_ref):
    d2 = d2_ref[0, 0, 0, :]
    dist = jnp.sqrt(d2)
    delta = _CUTOFF / (_G - 1)
    coeff = -0.5 / (delta * delta)
    offs = lax.broadcasted_iota(jnp.int32, (_G, 1), 0).astype(jnp.float32) * delta
    diff = dist[None, :] - offs
    eaT = jnp.exp(coeff * (diff * diff))
    a = _ssp(jnp.dot(w1T_ref[0], eaT,
                     preferred_element_type=jnp.float32) + b1_ref[0])
    wf = (jnp.dot(w2T_ref[0], a, preferred_element_type=jnp.float32)
          + b2_ref[0])
    cc = 0.5 * (jnp.cos(dist * (math.pi / _CUTOFF)) + 1.0)
    out_ref[0, 0] = wf * cc[None, :]


def _tc_wf(d2, w1T, b1c, w2T, b2c):
    d2r = d2.reshape(_K, _E // _BE, 1, _BE)
    return pl.pallas_call(
        _wf_body,
        grid=(_NI, _K, _E // _BE),
        in_specs=[
            pl.BlockSpec((1, 1, 1, _BE), lambda i, k, e: (k, e, 0, 0)),
            pl.BlockSpec((1, _FH, _G), lambda i, k, e: (i, 0, 0)),
            pl.BlockSpec((1, _FH, 1), lambda i, k, e: (i, 0, 0)),
            pl.BlockSpec((1, _FH, _FH), lambda i, k, e: (i, 0, 0)),
            pl.BlockSpec((1, _FH, 1), lambda i, k, e: (i, 0, 0)),
        ],
        out_specs=pl.BlockSpec((1, 1, _FH, _BE), lambda i, k, e: (i, k, 0, e)),
        out_shape=jax.ShapeDtypeStruct((_NI, _K, _FH, _E), jnp.float32),
    )(d2r, w1T, b1c, w2T, b2c)


def _embed_body(zf_ref, embT_ref, out_ref):
    zrow = zf_ref[0, 0, :]
    ids = lax.broadcasted_iota(jnp.int32, (100, 1), 0).astype(jnp.float32)
    oh = (zrow[None, :] == ids).astype(jnp.float32)
    out_ref[...] = jnp.dot(embT_ref[...], oh,
                           preferred_element_type=jnp.float32)


def _tc_embed(zf3, embT):
    return pl.pallas_call(
        _embed_body,
        grid=(_NP // _BN0,),
        in_specs=[
            pl.BlockSpec((1, 1, _BN0), lambda b: (b, 0, 0)),
            pl.BlockSpec((_H, 100), lambda b: (0, 0)),
        ],
        out_specs=pl.BlockSpec((_H, _BN0), lambda b: (0, b)),
        out_shape=jax.ShapeDtypeStruct((_H, _NP), jnp.float32),
    )(zf3, embT)


def _x1_body(hT_ref, w_ref, out_ref):
    out_ref[0] = jnp.dot(w_ref[...], hT_ref[0],
                         preferred_element_type=jnp.float32)


def _tc_x1(hT, l1T_i):
    return pl.pallas_call(
        _x1_body,
        grid=(_K, _NP // _BN),
        in_specs=[
            pl.BlockSpec((1, _H, _BN), lambda k, n: (k, 0, n)),
            pl.BlockSpec((_FH, _H), lambda k, n: (0, 0)),
        ],
        out_specs=pl.BlockSpec((1, _FH, _BN), lambda k, n: (k, 0, n)),
        out_shape=jax.ShapeDtypeStruct((_K, _FH, _NP), jnp.float32),
    )(hT, l1T_i)


def _upd_body(aggT_ref, hT_ref, w2T_ref, b2_ref, wT_ref, b_ref, out_ref):
    a = _ssp(jnp.dot(w2T_ref[...], aggT_ref[0],
                     preferred_element_type=jnp.float32) + b2_ref[...])
    x2 = jnp.dot(wT_ref[...], a,
                 preferred_element_type=jnp.float32) + b_ref[...]
    out_ref[0] = hT_ref[0] + x2


def _tc_update(aggT, hT, l2T_i, b2c_i, lT_i, bc_i):
    return pl.pallas_call(
        _upd_body,
        grid=(_K, _NP // _BN),
        in_specs=[
            pl.BlockSpec((1, _FH, _BN), lambda k, n: (k, 0, n)),
            pl.BlockSpec((1, _H, _BN), lambda k, n: (k, 0, n)),
            pl.BlockSpec((_H, _FH), lambda k, n: (0, 0)),
            pl.BlockSpec((_H, 1), lambda k, n: (0, 0)),
            pl.BlockSpec((_H, _H), lambda k, n: (0, 0)),
            pl.BlockSpec((_H, 1), lambda k, n: (0, 0)),
        ],
        out_specs=pl.BlockSpec((1, _H, _BN), lambda k, n: (k, 0, n)),
        out_shape=jax.ShapeDtypeStruct((_K, _H, _NP), jnp.float32),
    )(aggT, hT, l2T_i, b2c_i, lT_i, bc_i)


def _mean_body(hT_ref, out_ref):
    x = hT_ref[0]
    msk = lax.broadcasted_iota(jnp.int32, (1, _NP), 1) < _N
    out_ref[0, 0] = jnp.sum(jnp.where(msk, x, 0.0), axis=1) * (1.0 / _N)


def _tc_mean(hT):
    return pl.pallas_call(
        _mean_body,
        grid=(_K,),
        in_specs=[pl.BlockSpec((1, _H, _NP), lambda k: (k, 0, 0))],
        out_specs=pl.BlockSpec((1, 1, _H), lambda k: (k, 0, 0)),
        out_shape=jax.ShapeDtypeStruct((_K, 1, _H), jnp.float32),
    )(hT)


def _head_body(r_ref, w1_ref, b1_ref, w2T_ref, b2_ref,
               cw1_ref, cb1_ref, cw2_ref, cb2_ref, out_ref):
    r = r_ref[...]
    t = jnp.tanh(jnp.dot(r, w1_ref[...],
                         preferred_element_type=jnp.float32) + b1_ref[...])
    sc = jnp.sum(t * w2T_ref[...], axis=1, keepdims=True) + b2_ref[...]
    m = jnp.max(sc, axis=0, keepdims=True)
    e = jnp.exp(sc - m)
    w = e / jnp.sum(e, axis=0, keepdims=True)
    fused = jnp.sum(w * r, axis=0, keepdims=True)
    hid = jnp.maximum(
        jnp.dot(fused, cw1_ref[...],
                preferred_element_type=jnp.float32) + cb1_ref[...], 0.0)
    out_ref[...] = jnp.dot(hid, cw2_ref[...],
                           preferred_element_type=jnp.float32) + cb2_ref[...]


def _tc_head(reprs, attn_w1, attn_b1r, attn_w2T, attn_b2r,
             cls_w1, cls_b1r, cls_w2, cls_b2r):
    return pl.pallas_call(
        _head_body,
        out_shape=jax.ShapeDtypeStruct((1, _T), jnp.float32),
    )(reprs, attn_w1, attn_b1r, attn_w2T, attn_b2r,
      cls_w1, cls_b1r, cls_w2, cls_b2r)


def kernel(z, pos, edge_index, emb, mlp_w1, mlp_b1, mlp_w2, mlp_b2,
           lin1_w, lin2_w, lin2_b, lin_w, lin_b,
           attn_w1, attn_b1, attn_w2, attn_b2,
           cls_w1, cls_b1, cls_w2, cls_b2):
    posT = jnp.pad(jnp.transpose(pos, (0, 2, 1)),
                   ((0, 0), (0, 0), (0, _NP - _N))).reshape(-1)
    eidx = edge_index.astype(jnp.int32).reshape(-1)

    d2 = _sc_dist(posT, eidx).reshape(_K, _E)

    w1T = jnp.transpose(mlp_w1, (0, 2, 1))
    w2T = jnp.transpose(mlp_w2, (0, 2, 1))
    wfT = _tc_wf(d2, w1T, mlp_b1[:, :, None], w2T, mlp_b2[:, :, None])

    zf3 = jnp.pad(z.astype(jnp.float32),
                  (0, _NP - _N)).reshape(_NP // _BN0, 1, _BN0)
    hT0 = _tc_embed(zf3, emb.T)
    hT = jnp.broadcast_to(hT0[None], (_K, _H, _NP))

    for i in range(_NI):
        x1T = _tc_x1(hT, jnp.transpose(lin1_w[i]))
        aggT = _sc_scatter(x1T.reshape(-1), wfT[i].reshape(-1),
                           eidx).reshape(_K, _FH, _NP)
        hT = _tc_update(aggT, hT,
                        jnp.transpose(lin2_w[i]), lin2_b[i][:, None],
                        jnp.transpose(lin_w[i]), lin_b[i][:, None])

    reprs = _tc_mean(hT).reshape(_K, _H)
    out = _tc_head(reprs, attn_w1, attn_b1.reshape(1, _FH),
                   jnp.transpose(attn_w2), attn_b2.reshape(1, 1),
                   cls_w1, cls_b1.reshape(1, _H),
                   cls_w2, cls_b2.reshape(1, _T))
    return out.reshape(_T)


# speedup vs baseline: 4.2643x; 1.3335x over previous
import functools
import math

import jax
import jax.numpy as jnp
from jax import lax
from jax.experimental import pallas as pl
from jax.experimental.pallas import tpu as pltpu
from jax.experimental.pallas import tpu_sc as plsc

_K = 4
_N = 10000
_NP = 10240
_E = 160000
_H = 128
_FH = 64
_G = 50
_NI = 3
_CUTOFF = 10.0
_T = 12

_NTILES = 32
_CE = 10000
_BE = 6400
_BN = 2048
_BN0 = 1024


def _ssp(x):
    return (jnp.maximum(x, 0.0) + jnp.log1p(jnp.exp(-jnp.abs(x)))
            - math.log(2.0))


def _sc_dist_body(posT_hbm, eidx_hbm, d2_hbm, posb, srcb, dstb, outb):
    c = lax.axis_index("c")
    s = lax.axis_index("s")
    wid = s * 2 + c
    k = wid // 8
    ebase = (wid % 8) * (_E // 8)
    for r in range(3):
        pltpu.sync_copy(posT_hbm.at[pl.ds((k * 3 + r) * _NP, _NP)],
                        posb.at[pl.ds(r * _NP, _NP)])

    def chunk(ci, carry):
        off = ebase + ci * _CE
        pltpu.sync_copy(eidx_hbm.at[pl.ds(k * 2 * _E + off, _CE)], srcb)
        pltpu.sync_copy(eidx_hbm.at[pl.ds((k * 2 + 1) * _E + off, _CE)], dstb)

        @plsc.parallel_loop(0, _CE // 16, unroll=8)
        def inner(j):
            sv = srcb[pl.ds(j * 16, 16)]
            dv = dstb[pl.ds(j * 16, 16)]
            dx = (plsc.load_gather(posb, [sv])
                  - plsc.load_gather(posb, [dv]))
            dy = (plsc.load_gather(posb, [sv + _NP])
                  - plsc.load_gather(posb, [dv + _NP]))
            dz = (plsc.load_gather(posb, [sv + 2 * _NP])
                  - plsc.load_gather(posb, [dv + 2 * _NP]))
            outb[pl.ds(j * 16, 16)] = dx * dx + dy * dy + dz * dz
        pltpu.sync_copy(outb, d2_hbm.at[pl.ds(k * _E + off, _CE)])
        return carry

    lax.fori_loop(0, (_E // 8) // _CE, chunk, 0)


def _sc_dist(posT, eidx):
    mesh = plsc.VectorSubcoreMesh(core_axis_name="c", subcore_axis_name="s")
    return pl.kernel(
        _sc_dist_body,
        out_type=jax.ShapeDtypeStruct((_K * _E,), jnp.float32),
        mesh=mesh,
        compiler_params=pltpu.CompilerParams(needs_layout_passes=False),
        scratch_types=[
            pltpu.VMEM((3 * _NP,), jnp.float32),
            pltpu.VMEM((_CE,), jnp.int32),
            pltpu.VMEM((_CE,), jnp.int32),
            pltpu.VMEM((_CE,), jnp.float32),
        ],
    )(posT, eidx)


_NCH = _E // _CE


def _sc_scatter_body(x1T_hbm, wfT_hbm, eidx_hbm, aggT_hbm,
                     x1b, aggb,
                     srcA, dstA, wfA0, wfA1,
                     srcB, dstB, wfB0, wfB1, semA, semB):
    c = lax.axis_index("c")
    s = lax.axis_index("s")
    wid = s * 2 + c
    f0 = wid * 2
    zero16 = jnp.zeros((16,), jnp.float32)

    for k in range(_K):
        ebase = k * 2 * _E
        fbase = (k * _FH + f0) * _E
        nb = (k * _FH + f0) * _NP

        def start(ci, srcX, dstX, wfX0, wfX1, sem):
            off = ci * _CE
            pltpu.async_copy(eidx_hbm.at[pl.ds(ebase + off, _CE)], srcX, sem)
            pltpu.async_copy(eidx_hbm.at[pl.ds(ebase + _E + off, _CE)],
                             dstX, sem)
            pltpu.async_copy(wfT_hbm.at[pl.ds(fbase + off, _CE)], wfX0, sem)
            pltpu.async_copy(wfT_hbm.at[pl.ds(fbase + _E + off, _CE)],
                             wfX1, sem)

        def drain(ci, srcX, dstX, wfX0, wfX1, sem):
            off = ci * _CE
            pltpu.make_async_copy(eidx_hbm.at[pl.ds(ebase + off, _CE)],
                                  srcX, sem).wait()
            pltpu.make_async_copy(eidx_hbm.at[pl.ds(ebase + _E + off, _CE)],
                                  dstX, sem).wait()
            pltpu.make_async_copy(wfT_hbm.at[pl.ds(fbase + off, _CE)],
                                  wfX0, sem).wait()
            pltpu.make_async_copy(wfT_hbm.at[pl.ds(fbase + _E + off, _CE)],
                                  wfX1, sem).wait()

        def compute(srcX, dstX, wfX0, wfX1):
            @plsc.parallel_loop(0, _CE // 16, unroll=8)
            def inner(j):
                sv = srcX[pl.ds(j * 16, 16)]
                dv = dstX[pl.ds(j * 16, 16)]
                g0 = plsc.load_gather(x1b, [sv])
                g1 = plsc.load_gather(x1b, [sv + _NP])
                m0 = g0 * wfX0[pl.ds(j * 16, 16)]
                m1 = g1 * wfX1[pl.ds(j * 16, 16)]
                plsc.addupdate_scatter(aggb, [dv], m0)
                plsc.addupdate_scatter(aggb, [dv + _NP], m1)

        start(0, srcA, dstA, wfA0, wfA1, semA)

        @plsc.parallel_loop(0, (2 * _NP) // 16, unroll=8)
        def zbody(j):
            aggb[pl.ds(j * 16, 16)] = zero16

        pltpu.sync_copy(x1T_hbm.at[pl.ds(nb, _NP)], x1b.at[pl.ds(0, _NP)])
        pltpu.sync_copy(x1T_hbm.at[pl.ds(nb + _NP, _NP)],
                        x1b.at[pl.ds(_NP, _NP)])

        def pair(ci2, carry):
            ciA = 2 * ci2
            start(ciA + 1, srcB, dstB, wfB0, wfB1, semB)
            drain(ciA, srcA, dstA, wfA0, wfA1, semA)
            compute(srcA, dstA, wfA0, wfA1)

            @pl.when(ci2 < _NCH // 2 - 1)
            def _():
                start(ciA + 2, srcA, dstA, wfA0, wfA1, semA)

            drain(ciA + 1, srcB, dstB, wfB0, wfB1, semB)
            compute(srcB, dstB, wfB0, wfB1)
            return carry

        lax.fori_loop(0, _NCH // 2, pair, 0)

        pltpu.sync_copy(aggb.at[pl.ds(0, _NP)], aggT_hbm.at[pl.ds(nb, _NP)])
        pltpu.sync_copy(aggb.at[pl.ds(_NP, _NP)],
                        aggT_hbm.at[pl.ds(nb + _NP, _NP)])


def _sc_scatter(x1T, wfT_i, eidx):
    mesh = plsc.VectorSubcoreMesh(core_axis_name="c", subcore_axis_name="s")
    return pl.kernel(
        _sc_scatter_body,
        out_type=jax.ShapeDtypeStruct((_K * _FH * _NP,), jnp.float32),
        mesh=mesh,
        compiler_params=pltpu.CompilerParams(needs_layout_passes=False),
        scratch_types=[
            pltpu.VMEM((2 * _NP,), jnp.float32),
            pltpu.VMEM((2 * _NP,), jnp.float32),
            pltpu.VMEM((_CE,), jnp.int32),
            pltpu.VMEM((_CE,), jnp.int32),
            pltpu.VMEM((_CE,), jnp.float32),
            pltpu.VMEM((_CE,), jnp.float32),
            pltpu.VMEM((_CE,), jnp.int32),
            pltpu.VMEM((_CE,), jnp.int32),
            pltpu.VMEM((_CE,), jnp.float32),
            pltpu.VMEM((_CE,), jnp.float32),
            pltpu.SemaphoreType.DMA,
            pltpu.SemaphoreType.DMA,
        ],
    )(x1T, wfT_i, eidx)


def _wf_body(d2_ref, w1T_ref, b1_ref, w2T_ref, b2_ref, out_ref):
    d2 = d2_ref[0, 0, 0, :]
    dist = jnp.sqrt(d2)
    delta = _CUTOFF / (_G - 1)
    coeff = -0.5 / (delta * delta)
    offs = lax.broadcasted_iota(jnp.int32, (_G, 1), 0).astype(jnp.float32) * delta
    diff = dist[None, :] - offs
    eaT = jnp.exp(coeff * (diff * diff))
    a = _ssp(jnp.dot(w1T_ref[0], eaT,
                     preferred_element_type=jnp.float32) + b1_ref[0])
    wf = (jnp.dot(w2T_ref[0], a, preferred_element_type=jnp.float32)
          + b2_ref[0])
    cc = 0.5 * (jnp.cos(dist * (math.pi / _CUTOFF)) + 1.0)
    out_ref[0, 0] = wf * cc[None, :]


def _tc_wf(d2, w1T, b1c, w2T, b2c):
    d2r = d2.reshape(_K, _E // _BE, 1, _BE)
    return pl.pallas_call(
        _wf_body,
        grid=(_NI, _K, _E // _BE),
        in_specs=[
            pl.BlockSpec((1, 1, 1, _BE), lambda i, k, e: (k, e, 0, 0)),
            pl.BlockSpec((1, _FH, _G), lambda i, k, e: (i, 0, 0)),
            pl.BlockSpec((1, _FH, 1), lambda i, k, e: (i, 0, 0)),
            pl.BlockSpec((1, _FH, _FH), lambda i, k, e: (i, 0, 0)),
            pl.BlockSpec((1, _FH, 1), lambda i, k, e: (i, 0, 0)),
        ],
        out_specs=pl.BlockSpec((1, 1, _FH, _BE), lambda i, k, e: (i, k, 0, e)),
        out_shape=jax.ShapeDtypeStruct((_NI, _K, _FH, _E), jnp.float32),
    )(d2r, w1T, b1c, w2T, b2c)


def _embed_body(zf_ref, embT_ref, out_ref):
    zrow = zf_ref[0, 0, :]
    ids = lax.broadcasted_iota(jnp.int32, (100, 1), 0).astype(jnp.float32)
    oh = (zrow[None, :] == ids).astype(jnp.float32)
    out_ref[...] = jnp.dot(embT_ref[...], oh,
                           preferred_element_type=jnp.float32)


def _tc_embed(zf3, embT):
    return pl.pallas_call(
        _embed_body,
        grid=(_NP // _BN0,),
        in_specs=[
            pl.BlockSpec((1, 1, _BN0), lambda b: (b, 0, 0)),
            pl.BlockSpec((_H, 100), lambda b: (0, 0)),
        ],
        out_specs=pl.BlockSpec((_H, _BN0), lambda b: (0, b)),
        out_shape=jax.ShapeDtypeStruct((_H, _NP), jnp.float32),
    )(zf3, embT)


def _x1_body(hT_ref, w_ref, out_ref):
    out_ref[0] = jnp.dot(w_ref[...], hT_ref[0],
                         preferred_element_type=jnp.float32)


def _tc_x1(hT, l1T_i):
    return pl.pallas_call(
        _x1_body,
        grid=(_K, _NP // _BN),
        in_specs=[
            pl.BlockSpec((1, _H, _BN), lambda k, n: (k, 0, n)),
            pl.BlockSpec((_FH, _H), lambda k, n: (0, 0)),
        ],
        out_specs=pl.BlockSpec((1, _FH, _BN), lambda k, n: (k, 0, n)),
        out_shape=jax.ShapeDtypeStruct((_K, _FH, _NP), jnp.float32),
    )(hT, l1T_i)


def _upd_body(aggT_ref, hT_ref, w2T_ref, b2_ref, wT_ref, b_ref, out_ref):
    a = _ssp(jnp.dot(w2T_ref[...], aggT_ref[0],
                     preferred_element_type=jnp.float32) + b2_ref[...])
    x2 = jnp.dot(wT_ref[...], a,
                 preferred_element_type=jnp.float32) + b_ref[...]
    out_ref[0] = hT_ref[0] + x2


def _tc_update(aggT, hT, l2T_i, b2c_i, lT_i, bc_i):
    return pl.pallas_call(
        _upd_body,
        grid=(_K, _NP // _BN),
        in_specs=[
            pl.BlockSpec((1, _FH, _BN), lambda k, n: (k, 0, n)),
            pl.BlockSpec((1, _H, _BN), lambda k, n: (k, 0, n)),
            pl.BlockSpec((_H, _FH), lambda k, n: (0, 0)),
            pl.BlockSpec((_H, 1), lambda k, n: (0, 0)),
            pl.BlockSpec((_H, _H), lambda k, n: (0, 0)),
            pl.BlockSpec((_H, 1), lambda k, n: (0, 0)),
        ],
        out_specs=pl.BlockSpec((1, _H, _BN), lambda k, n: (k, 0, n)),
        out_shape=jax.ShapeDtypeStruct((_K, _H, _NP), jnp.float32),
    )(aggT, hT, l2T_i, b2c_i, lT_i, bc_i)


def _mean_body(hT_ref, out_ref):
    x = hT_ref[0]
    msk = lax.broadcasted_iota(jnp.int32, (1, _NP), 1) < _N
    out_ref[0, 0] = jnp.sum(jnp.where(msk, x, 0.0), axis=1) * (1.0 / _N)


def _tc_mean(hT):
    return pl.pallas_call(
        _mean_body,
        grid=(_K,),
        in_specs=[pl.BlockSpec((1, _H, _NP), lambda k: (k, 0, 0))],
        out_specs=pl.BlockSpec((1, 1, _H), lambda k: (k, 0, 0)),
        out_shape=jax.ShapeDtypeStruct((_K, 1, _H), jnp.float32),
    )(hT)


def _head_body(r_ref, w1_ref, b1_ref, w2T_ref, b2_ref,
               cw1_ref, cb1_ref, cw2_ref, cb2_ref, out_ref):
    r = r_ref[...]
    t = jnp.tanh(jnp.dot(r, w1_ref[...],
                         preferred_element_type=jnp.float32) + b1_ref[...])
    sc = jnp.sum(t * w2T_ref[...], axis=1, keepdims=True) + b2_ref[...]
    m = jnp.max(sc, axis=0, keepdims=True)
    e = jnp.exp(sc - m)
    w = e / jnp.sum(e, axis=0, keepdims=True)
    fused = jnp.sum(w * r, axis=0, keepdims=True)
    hid = jnp.maximum(
        jnp.dot(fused, cw1_ref[...],
                preferred_element_type=jnp.float32) + cb1_ref[...], 0.0)
    out_ref[...] = jnp.dot(hid, cw2_ref[...],
                           preferred_element_type=jnp.float32) + cb2_ref[...]


def _tc_head(reprs, attn_w1, attn_b1r, attn_w2T, attn_b2r,
             cls_w1, cls_b1r, cls_w2, cls_b2r):
    return pl.pallas_call(
        _head_body,
        out_shape=jax.ShapeDtypeStruct((1, _T), jnp.float32),
    )(reprs, attn_w1, attn_b1r, attn_w2T, attn_b2r,
      cls_w1, cls_b1r, cls_w2, cls_b2r)


def kernel(z, pos, edge_index, emb, mlp_w1, mlp_b1, mlp_w2, mlp_b2,
           lin1_w, lin2_w, lin2_b, lin_w, lin_b,
           attn_w1, attn_b1, attn_w2, attn_b2,
           cls_w1, cls_b1, cls_w2, cls_b2):
    posT = jnp.pad(jnp.transpose(pos, (0, 2, 1)),
                   ((0, 0), (0, 0), (0, _NP - _N))).reshape(-1)
    eidx = edge_index.astype(jnp.int32).reshape(-1)

    d2 = _sc_dist(posT, eidx).reshape(_K, _E)

    w1T = jnp.transpose(mlp_w1, (0, 2, 1))
    w2T = jnp.transpose(mlp_w2, (0, 2, 1))
    wfT = _tc_wf(d2, w1T, mlp_b1[:, :, None], w2T, mlp_b2[:, :, None])

    zf3 = jnp.pad(z.astype(jnp.float32),
                  (0, _NP - _N)).reshape(_NP // _BN0, 1, _BN0)
    hT0 = _tc_embed(zf3, emb.T)
    hT = jnp.broadcast_to(hT0[None], (_K, _H, _NP))

    for i in range(_NI):
        x1T = _tc_x1(hT, jnp.transpose(lin1_w[i]))
        aggT = _sc_scatter(x1T.reshape(-1), wfT[i].reshape(-1),
                           eidx).reshape(_K, _FH, _NP)
        hT = _tc_update(aggT, hT,
                        jnp.transpose(lin2_w[i]), lin2_b[i][:, None],
                        jnp.transpose(lin_w[i]), lin_b[i][:, None])

    reprs = _tc_mean(hT).reshape(_K, _H)
    out = _tc_head(reprs, attn_w1, attn_b1.reshape(1, _FH),
                   jnp.transpose(attn_w2), attn_b2.reshape(1, 1),
                   cls_w1, cls_b1.reshape(1, _H),
                   cls_w2, cls_b2.reshape(1, _T))
    return out.reshape(_T)


# speedup vs baseline: 6.1838x; 1.4502x over previous
import functools
import math

import jax
import jax.numpy as jnp
from jax import lax
from jax.experimental import pallas as pl
from jax.experimental.pallas import tpu as pltpu
from jax.experimental.pallas import tpu_sc as plsc

_K = 4
_N = 10000
_NP = 10240
_E = 160000
_H = 128
_FH = 64
_G = 50
_NI = 3
_CUTOFF = 10.0
_T = 12

_NTILES = 32
_CE = 6400
_CED = 4000
_BE = 6400
_BN = 2048
_BN0 = 1024


def _ssp(x):
    return (jnp.maximum(x, 0.0) + jnp.log1p(jnp.exp(-jnp.abs(x)))
            - math.log(2.0))


def _sc_dist_body(posT_hbm, eidx_hbm, d2_hbm, posb, srcb, dstb, outb):
    c = lax.axis_index("c")
    s = lax.axis_index("s")
    wid = s * 2 + c
    k = wid // 8
    ebase = (wid % 8) * (_E // 8)
    for r in range(3):
        pltpu.sync_copy(posT_hbm.at[pl.ds((k * 3 + r) * _NP, _NP)],
                        posb.at[pl.ds(r * _NP, _NP)])

    def chunk(ci, carry):
        off = ebase + ci * _CED
        pltpu.sync_copy(eidx_hbm.at[pl.ds(k * 2 * _E + off, _CED)], srcb)
        pltpu.sync_copy(eidx_hbm.at[pl.ds((k * 2 + 1) * _E + off, _CED)], dstb)

        @plsc.parallel_loop(0, _CED // 16, unroll=8)
        def inner(j):
            sv = srcb[pl.ds(j * 16, 16)]
            dv = dstb[pl.ds(j * 16, 16)]
            dx = (plsc.load_gather(posb, [sv])
                  - plsc.load_gather(posb, [dv]))
            dy = (plsc.load_gather(posb, [sv + _NP])
                  - plsc.load_gather(posb, [dv + _NP]))
            dz = (plsc.load_gather(posb, [sv + 2 * _NP])
                  - plsc.load_gather(posb, [dv + 2 * _NP]))
            outb[pl.ds(j * 16, 16)] = dx * dx + dy * dy + dz * dz
        pltpu.sync_copy(outb, d2_hbm.at[pl.ds(k * _E + off, _CED)])
        return carry

    lax.fori_loop(0, (_E // 8) // _CED, chunk, 0)


def _sc_dist(posT, eidx):
    mesh = plsc.VectorSubcoreMesh(core_axis_name="c", subcore_axis_name="s")
    return pl.kernel(
        _sc_dist_body,
        out_type=jax.ShapeDtypeStruct((_K * _E,), jnp.float32),
        mesh=mesh,
        compiler_params=pltpu.CompilerParams(needs_layout_passes=False),
        scratch_types=[
            pltpu.VMEM((3 * _NP,), jnp.float32),
            pltpu.VMEM((_CED,), jnp.int32),
            pltpu.VMEM((_CED,), jnp.int32),
            pltpu.VMEM((_CED,), jnp.float32),
        ],
    )(posT, eidx)


_NCH = _E // _CE


def _sc_scatter_body(i, x1T_hbm, wfT_hbm, eidx_hbm, aggT_hbm,
                     x1b, aggb,
                     srcA, dstA, wfA0, wfA1,
                     srcB, dstB, wfB0, wfB1, semA, semB):
    c = lax.axis_index("c")
    s = lax.axis_index("s")
    wid = s * 2 + c
    f0 = wid * 2
    zero16 = jnp.zeros((16,), jnp.float32)

    for k in range(_K):
        ebase = k * 2 * _E
        frow = (i * _K + k) * _FH + f0
        nrow = k * _FH + f0

        def start(ci, srcX, dstX, wfX0, wfX1, sem):
            off = ci * _CE
            pltpu.async_copy(eidx_hbm.at[pl.ds(ebase + off, _CE)], srcX, sem)
            pltpu.async_copy(eidx_hbm.at[pl.ds(ebase + _E + off, _CE)],
                             dstX, sem)
            pltpu.async_copy(wfT_hbm.at[frow, 0, pl.ds(off, _CE)], wfX0,
                             sem)
            pltpu.async_copy(wfT_hbm.at[frow + 1, 0, pl.ds(off, _CE)],
                             wfX1, sem)

        def drain(ci, srcX, dstX, wfX0, wfX1, sem):
            off = ci * _CE
            pltpu.make_async_copy(eidx_hbm.at[pl.ds(ebase + off, _CE)],
                                  srcX, sem).wait()
            pltpu.make_async_copy(eidx_hbm.at[pl.ds(ebase + _E + off, _CE)],
                                  dstX, sem).wait()
            pltpu.make_async_copy(wfT_hbm.at[frow, 0, pl.ds(off, _CE)],
                                  wfX0, sem).wait()
            pltpu.make_async_copy(wfT_hbm.at[frow + 1, 0, pl.ds(off, _CE)],
                                  wfX1, sem).wait()

        def compute(srcX, dstX, wfX0, wfX1):
            @plsc.parallel_loop(0, _CE // 16, unroll=8)
            def inner(j):
                sv = srcX[pl.ds(j * 16, 16)]
                dv = dstX[pl.ds(j * 16, 16)]
                g0 = plsc.load_gather(x1b, [sv])
                g1 = plsc.load_gather(x1b, [sv + _NP])
                m0 = g0 * wfX0[pl.ds(j * 16, 16)]
                m1 = g1 * wfX1[pl.ds(j * 16, 16)]
                plsc.addupdate_scatter(aggb, [dv], m0)
                plsc.addupdate_scatter(aggb, [dv + _NP], m1)

        start(0, srcA, dstA, wfA0, wfA1, semA)

        @plsc.parallel_loop(0, (2 * _NP) // 16, unroll=8)
        def zbody(j):
            aggb[pl.ds(j * 16, 16)] = zero16

        pltpu.sync_copy(x1T_hbm.at[nrow, 0], x1b.at[pl.ds(0, _NP)])
        pltpu.sync_copy(x1T_hbm.at[nrow + 1, 0], x1b.at[pl.ds(_NP, _NP)])

        def pair(ci2, carry):
            ciA = 2 * ci2
            start(ciA + 1, srcB, dstB, wfB0, wfB1, semB)
            drain(ciA, srcA, dstA, wfA0, wfA1, semA)
            compute(srcA, dstA, wfA0, wfA1)
            start(ciA + 2, srcA, dstA, wfA0, wfA1, semA)
            drain(ciA + 1, srcB, dstB, wfB0, wfB1, semB)
            compute(srcB, dstB, wfB0, wfB1)
            return carry

        lax.fori_loop(0, _NCH // 2, pair, 0)
        drain(_NCH - 1, srcA, dstA, wfA0, wfA1, semA)
        compute(srcA, dstA, wfA0, wfA1)

        pltpu.sync_copy(aggb.at[pl.ds(0, _NP)], aggT_hbm.at[nrow, 0])
        pltpu.sync_copy(aggb.at[pl.ds(_NP, _NP)], aggT_hbm.at[nrow + 1, 0])


def _sc_scatter(x1T, wfT, eidx, i):
    mesh = plsc.VectorSubcoreMesh(core_axis_name="c", subcore_axis_name="s")
    return pl.kernel(
        functools.partial(_sc_scatter_body, i),
        out_type=jax.ShapeDtypeStruct((_K * _FH, 1, _NP), jnp.float32),
        mesh=mesh,
        compiler_params=pltpu.CompilerParams(needs_layout_passes=False),
        scratch_types=[
            pltpu.VMEM((2 * _NP,), jnp.float32),
            pltpu.VMEM((2 * _NP,), jnp.float32),
            pltpu.VMEM((_CE,), jnp.int32),
            pltpu.VMEM((_CE,), jnp.int32),
            pltpu.VMEM((_CE,), jnp.float32),
            pltpu.VMEM((_CE,), jnp.float32),
            pltpu.VMEM((_CE,), jnp.int32),
            pltpu.VMEM((_CE,), jnp.int32),
            pltpu.VMEM((_CE,), jnp.float32),
            pltpu.VMEM((_CE,), jnp.float32),
            pltpu.SemaphoreType.DMA,
            pltpu.SemaphoreType.DMA,
        ],
    )(x1T, wfT, eidx)


def _wf_body(d2_ref, w1T_ref, b1_ref, w2T_ref, b2_ref, out_ref):
    d2 = d2_ref[0, 0, :]
    dist = jnp.sqrt(d2)
    delta = _CUTOFF / (_G - 1)
    coeff = -0.5 / (delta * delta)
    offs = lax.broadcasted_iota(jnp.int32, (_G, 1), 0).astype(jnp.float32) * delta
    diff = dist[None, :] - offs
    eaT = jnp.exp(coeff * (diff * diff))
    a = _ssp(jnp.dot(w1T_ref[0], eaT,
                     preferred_element_type=jnp.float32) + b1_ref[0])
    wf = (jnp.dot(w2T_ref[0], a, preferred_element_type=jnp.float32)
          + b2_ref[0])
    cc = 0.5 * (jnp.cos(dist * (math.pi / _CUTOFF)) + 1.0)
    out_ref[:, 0, :] = wf * cc[None, :]


def _tc_wf(d2, w1T, b1c, w2T, b2c):
    d2 = d2.reshape(_K, 1, _E)
    return pl.pallas_call(
        _wf_body,
        grid=(_NI, _K, _E // _BE),
        in_specs=[
            pl.BlockSpec((1, 1, _BE), lambda i, k, e: (k, 0, e)),
            pl.BlockSpec((1, _FH, _G), lambda i, k, e: (i, 0, 0)),
            pl.BlockSpec((1, _FH, 1), lambda i, k, e: (i, 0, 0)),
            pl.BlockSpec((1, _FH, _FH), lambda i, k, e: (i, 0, 0)),
            pl.BlockSpec((1, _FH, 1), lambda i, k, e: (i, 0, 0)),
        ],
        out_specs=pl.BlockSpec((_FH, 1, _BE),
                               lambda i, k, e: (i * _K + k, 0, e)),
        out_shape=jax.ShapeDtypeStruct((_NI * _K * _FH, 1, _E),
                                       jnp.float32),
    )(d2, w1T, b1c, w2T, b2c)


def _embed_body(zf_ref, embT_ref, out_ref):
    zrow = zf_ref[0, 0, :]
    ids = lax.broadcasted_iota(jnp.int32, (100, 1), 0).astype(jnp.float32)
    oh = (zrow[None, :] == ids).astype(jnp.float32)
    out_ref[...] = jnp.dot(embT_ref[...], oh,
                           preferred_element_type=jnp.float32)


def _tc_embed(zf3, embT):
    return pl.pallas_call(
        _embed_body,
        grid=(_NP // _BN0,),
        in_specs=[
            pl.BlockSpec((1, 1, _BN0), lambda b: (b, 0, 0)),
            pl.BlockSpec((_H, 100), lambda b: (0, 0)),
        ],
        out_specs=pl.BlockSpec((_H, _BN0), lambda b: (0, b)),
        out_shape=jax.ShapeDtypeStruct((_H, _NP), jnp.float32),
    )(zf3, embT)


def _x1_body(hT_ref, w_ref, out_ref):
    out_ref[:, 0, :] = jnp.dot(w_ref[...], hT_ref[0],
                               preferred_element_type=jnp.float32)


def _tc_x1(hT, l1T_i):
    return pl.pallas_call(
        _x1_body,
        grid=(_K, _NP // _BN),
        in_specs=[
            pl.BlockSpec((1, _H, _BN), lambda k, n: (k, 0, n)),
            pl.BlockSpec((_FH, _H), lambda k, n: (0, 0)),
        ],
        out_specs=pl.BlockSpec((_FH, 1, _BN), lambda k, n: (k, 0, n)),
        out_shape=jax.ShapeDtypeStruct((_K * _FH, 1, _NP), jnp.float32),
    )(hT, l1T_i)


def _upd_body(aggT_ref, hT_ref, w2T_ref, b2_ref, wT_ref, b_ref, out_ref):
    a = _ssp(jnp.dot(w2T_ref[...], aggT_ref[:, 0, :],
                     preferred_element_type=jnp.float32) + b2_ref[...])
    x2 = jnp.dot(wT_ref[...], a,
                 preferred_element_type=jnp.float32) + b_ref[...]
    out_ref[0] = hT_ref[0] + x2


def _tc_update(aggT, hT, l2T_i, b2c_i, lT_i, bc_i):
    return pl.pallas_call(
        _upd_body,
        grid=(_K, _NP // _BN),
        in_specs=[
            pl.BlockSpec((_FH, 1, _BN), lambda k, n: (k, 0, n)),
            pl.BlockSpec((1, _H, _BN), lambda k, n: (k, 0, n)),
            pl.BlockSpec((_H, _FH), lambda k, n: (0, 0)),
            pl.BlockSpec((_H, 1), lambda k, n: (0, 0)),
            pl.BlockSpec((_H, _H), lambda k, n: (0, 0)),
            pl.BlockSpec((_H, 1), lambda k, n: (0, 0)),
        ],
        out_specs=pl.BlockSpec((1, _H, _BN), lambda k, n: (k, 0, n)),
        out_shape=jax.ShapeDtypeStruct((_K, _H, _NP), jnp.float32),
    )(aggT, hT, l2T_i, b2c_i, lT_i, bc_i)


def _mean_body(hT_ref, out_ref):
    x = hT_ref[0]
    msk = lax.broadcasted_iota(jnp.int32, (1, _NP), 1) < _N
    out_ref[0, 0] = jnp.sum(jnp.where(msk, x, 0.0), axis=1) * (1.0 / _N)


def _tc_mean(hT):
    return pl.pallas_call(
        _mean_body,
        grid=(_K,),
        in_specs=[pl.BlockSpec((1, _H, _NP), lambda k: (k, 0, 0))],
        out_specs=pl.BlockSpec((1, 1, _H), lambda k: (k, 0, 0)),
        out_shape=jax.ShapeDtypeStruct((_K, 1, _H), jnp.float32),
    )(hT)


def _head_body(r_ref, w1_ref, b1_ref, w2T_ref, b2_ref,
               cw1_ref, cb1_ref, cw2_ref, cb2_ref, out_ref):
    r = r_ref[...]
    t = jnp.tanh(jnp.dot(r, w1_ref[...],
                         preferred_element_type=jnp.float32) + b1_ref[...])
    sc = jnp.sum(t * w2T_ref[...], axis=1, keepdims=True) + b2_ref[...]
    m = jnp.max(sc, axis=0, keepdims=True)
    e = jnp.exp(sc - m)
    w = e / jnp.sum(e, axis=0, keepdims=True)
    fused = jnp.sum(w * r, axis=0, keepdims=True)
    hid = jnp.maximum(
        jnp.dot(fused, cw1_ref[...],
                preferred_element_type=jnp.float32) + cb1_ref[...], 0.0)
    out_ref[...] = jnp.dot(hid, cw2_ref[...],
                           preferred_element_type=jnp.float32) + cb2_ref[...]


def _tc_head(reprs, attn_w1, attn_b1r, attn_w2T, attn_b2r,
             cls_w1, cls_b1r, cls_w2, cls_b2r):
    return pl.pallas_call(
        _head_body,
        out_shape=jax.ShapeDtypeStruct((1, _T), jnp.float32),
    )(reprs, attn_w1, attn_b1r, attn_w2T, attn_b2r,
      cls_w1, cls_b1r, cls_w2, cls_b2r)


def kernel(z, pos, edge_index, emb, mlp_w1, mlp_b1, mlp_w2, mlp_b2,
           lin1_w, lin2_w, lin2_b, lin_w, lin_b,
           attn_w1, attn_b1, attn_w2, attn_b2,
           cls_w1, cls_b1, cls_w2, cls_b2):
    posT = jnp.pad(jnp.transpose(pos, (0, 2, 1)),
                   ((0, 0), (0, 0), (0, _NP - _N))).reshape(-1)
    eidx = edge_index.astype(jnp.int32).reshape(-1)

    d2 = _sc_dist(posT, eidx)

    w1T = jnp.transpose(mlp_w1, (0, 2, 1))
    w2T = jnp.transpose(mlp_w2, (0, 2, 1))
    wfT = _tc_wf(d2, w1T, mlp_b1[:, :, None], w2T, mlp_b2[:, :, None])

    zf3 = jnp.pad(z.astype(jnp.float32),
                  (0, _NP - _N)).reshape(_NP // _BN0, 1, _BN0)
    hT0 = _tc_embed(zf3, emb.T)
    hT = jnp.broadcast_to(hT0[None], (_K, _H, _NP))

    for i in range(_NI):
        x1T = _tc_x1(hT, jnp.transpose(lin1_w[i]))
        aggT = _sc_scatter(x1T, wfT, eidx, i)
        hT = _tc_update(aggT, hT,
                        jnp.transpose(lin2_w[i]), lin2_b[i][:, None],
                        jnp.transpose(lin_w[i]), lin_b[i][:, None])

    reprs = _tc_mean(hT).reshape(_K, _H)
    out = _tc_head(reprs, attn_w1, attn_b1.reshape(1, _FH),
                   jnp.transpose(attn_w2), attn_b2.reshape(1, 1),
                   cls_w1, cls_b1.reshape(1, _H),
                   cls_w2, cls_b2.reshape(1, _T))
    return out.reshape(_T)


# speedup vs baseline: 8.0174x; 1.2965x over previous
import functools
import math

import jax
import jax.numpy as jnp
from jax import lax
from jax.experimental import pallas as pl
from jax.experimental.pallas import tpu as pltpu
from jax.experimental.pallas import tpu_sc as plsc

_K = 4
_N = 10000
_NP = 10240
_E = 160000
_H = 128
_FH = 64
_G = 50
_NI = 3
_CUTOFF = 10.0
_T = 12

_NTILES = 32
_CE = 6400
_CED = 4000
_BE = 6400
_BN = 2048
_BN0 = 1024


def _ssp(x):
    return (jnp.maximum(x, 0.0) + jnp.log1p(jnp.exp(-jnp.abs(x)))
            - math.log(2.0))


def _sc_dist_body(posT_hbm, eidx_hbm, d2_hbm, posb, srcb, dstb, outb):
    c = lax.axis_index("c")
    s = lax.axis_index("s")
    wid = s * 2 + c
    k = wid // 8
    ebase = (wid % 8) * (_E // 8)
    for r in range(3):
        pltpu.sync_copy(posT_hbm.at[pl.ds((k * 3 + r) * _NP, _NP)],
                        posb.at[pl.ds(r * _NP, _NP)])

    def chunk(ci, carry):
        off = ebase + ci * _CED
        pltpu.sync_copy(eidx_hbm.at[pl.ds(k * 2 * _E + off, _CED)], srcb)
        pltpu.sync_copy(eidx_hbm.at[pl.ds((k * 2 + 1) * _E + off, _CED)], dstb)

        @plsc.parallel_loop(0, _CED // 16, unroll=8)
        def inner(j):
            sv = srcb[pl.ds(j * 16, 16)]
            dv = dstb[pl.ds(j * 16, 16)]
            dx = (plsc.load_gather(posb, [sv])
                  - plsc.load_gather(posb, [dv]))
            dy = (plsc.load_gather(posb, [sv + _NP])
                  - plsc.load_gather(posb, [dv + _NP]))
            dz = (plsc.load_gather(posb, [sv + 2 * _NP])
                  - plsc.load_gather(posb, [dv + 2 * _NP]))
            outb[pl.ds(j * 16, 16)] = dx * dx + dy * dy + dz * dz
        pltpu.sync_copy(outb, d2_hbm.at[pl.ds(k * _E + off, _CED)])
        return carry

    lax.fori_loop(0, (_E // 8) // _CED, chunk, 0)


def _sc_dist(posT, eidx):
    mesh = plsc.VectorSubcoreMesh(core_axis_name="c", subcore_axis_name="s")
    return pl.kernel(
        _sc_dist_body,
        out_type=jax.ShapeDtypeStruct((_K * _E,), jnp.float32),
        mesh=mesh,
        compiler_params=pltpu.CompilerParams(needs_layout_passes=False),
        scratch_types=[
            pltpu.VMEM((3 * _NP,), jnp.float32),
            pltpu.VMEM((_CED,), jnp.int32),
            pltpu.VMEM((_CED,), jnp.int32),
            pltpu.VMEM((_CED,), jnp.float32),
        ],
    )(posT, eidx)


_NCH = _E // _CE


def _sc_scatter_body(x1T_hbm, wfT_hbm, eidx_hbm, aggT_hbm,
                     x1b, aggb,
                     srcA, dstA, wfA0, wfA1,
                     srcB, dstB, wfB0, wfB1, semA, semB):
    c = lax.axis_index("c")
    s = lax.axis_index("s")
    wid = s * 2 + c
    f0 = wid * 2
    zero16 = jnp.zeros((16,), jnp.float32)

    for k in range(_K):
        ebase = k * 2 * _E
        frow = k * _FH + f0
        nrow = frow

        def start(ci, srcX, dstX, wfX0, wfX1, sem):
            off = ci * _CE
            pltpu.async_copy(eidx_hbm.at[pl.ds(ebase + off, _CE)], srcX, sem)
            pltpu.async_copy(eidx_hbm.at[pl.ds(ebase + _E + off, _CE)],
                             dstX, sem)
            pltpu.async_copy(wfT_hbm.at[frow, 0, pl.ds(off, _CE)], wfX0,
                             sem)
            pltpu.async_copy(wfT_hbm.at[frow + 1, 0, pl.ds(off, _CE)],
                             wfX1, sem)

        def drain(ci, srcX, dstX, wfX0, wfX1, sem):
            off = ci * _CE
            pltpu.make_async_copy(eidx_hbm.at[pl.ds(ebase + off, _CE)],
                                  srcX, sem).wait()
            pltpu.make_async_copy(eidx_hbm.at[pl.ds(ebase + _E + off, _CE)],
                                  dstX, sem).wait()
            pltpu.make_async_copy(wfT_hbm.at[frow, 0, pl.ds(off, _CE)],
                                  wfX0, sem).wait()
            pltpu.make_async_copy(wfT_hbm.at[frow + 1, 0, pl.ds(off, _CE)],
                                  wfX1, sem).wait()

        def compute(srcX, dstX, wfX0, wfX1):
            @plsc.parallel_loop(0, _CE // 16, unroll=16)
            def inner(j):
                sv = srcX[pl.ds(j * 16, 16)]
                dv = dstX[pl.ds(j * 16, 16)]
                g0 = plsc.load_gather(x1b, [sv])
                g1 = plsc.load_gather(x1b, [sv + _NP])
                m0 = g0 * wfX0[pl.ds(j * 16, 16)]
                m1 = g1 * wfX1[pl.ds(j * 16, 16)]
                plsc.addupdate_scatter(aggb, [dv], m0)
                plsc.addupdate_scatter(aggb, [dv + _NP], m1)

        start(0, srcA, dstA, wfA0, wfA1, semA)

        @plsc.parallel_loop(0, (2 * _NP) // 16, unroll=8)
        def zbody(j):
            aggb[pl.ds(j * 16, 16)] = zero16

        pltpu.sync_copy(x1T_hbm.at[nrow, 0], x1b.at[pl.ds(0, _NP)])
        pltpu.sync_copy(x1T_hbm.at[nrow + 1, 0], x1b.at[pl.ds(_NP, _NP)])

        def pair(ci2, carry):
            ciA = 2 * ci2
            start(ciA + 1, srcB, dstB, wfB0, wfB1, semB)
            drain(ciA, srcA, dstA, wfA0, wfA1, semA)
            compute(srcA, dstA, wfA0, wfA1)
            start(ciA + 2, srcA, dstA, wfA0, wfA1, semA)
            drain(ciA + 1, srcB, dstB, wfB0, wfB1, semB)
            compute(srcB, dstB, wfB0, wfB1)
            return carry

        lax.fori_loop(0, _NCH // 2, pair, 0)
        drain(_NCH - 1, srcA, dstA, wfA0, wfA1, semA)
        compute(srcA, dstA, wfA0, wfA1)

        pltpu.sync_copy(aggb.at[pl.ds(0, _NP)], aggT_hbm.at[nrow, 0])
        pltpu.sync_copy(aggb.at[pl.ds(_NP, _NP)], aggT_hbm.at[nrow + 1, 0])


def _sc_scatter(x1T, wfT_i, eidx):
    mesh = plsc.VectorSubcoreMesh(core_axis_name="c", subcore_axis_name="s")
    return pl.kernel(
        _sc_scatter_body,
        out_type=jax.ShapeDtypeStruct((_K * _FH, 1, _NP), jnp.float32),
        mesh=mesh,
        compiler_params=pltpu.CompilerParams(needs_layout_passes=False),
        scratch_types=[
            pltpu.VMEM((2 * _NP,), jnp.float32),
            pltpu.VMEM((2 * _NP,), jnp.float32),
            pltpu.VMEM((_CE,), jnp.int32),
            pltpu.VMEM((_CE,), jnp.int32),
            pltpu.VMEM((_CE,), jnp.float32),
            pltpu.VMEM((_CE,), jnp.float32),
            pltpu.VMEM((_CE,), jnp.int32),
            pltpu.VMEM((_CE,), jnp.int32),
            pltpu.VMEM((_CE,), jnp.float32),
            pltpu.VMEM((_CE,), jnp.float32),
            pltpu.SemaphoreType.DMA,
            pltpu.SemaphoreType.DMA,
        ],
    )(x1T, wfT_i, eidx)


def _wf_body(d2_ref, w1T_ref, b1_ref, w2T_ref, b2_ref, out_ref):
    d2 = d2_ref[0, 0, :]
    dist = jnp.sqrt(d2)
    delta = _CUTOFF / (_G - 1)
    coeff = -0.5 / (delta * delta)
    offs = lax.broadcasted_iota(jnp.int32, (_G, 1), 0).astype(jnp.float32) * delta
    diff = dist[None, :] - offs
    eaT = jnp.exp(coeff * (diff * diff))
    a = _ssp(jnp.dot(w1T_ref[...], eaT,
                     preferred_element_type=jnp.float32) + b1_ref[...])
    wf = (jnp.dot(w2T_ref[...], a, preferred_element_type=jnp.float32)
          + b2_ref[...])
    cc = 0.5 * (jnp.cos(dist * (math.pi / _CUTOFF)) + 1.0)
    out_ref[:, 0, :] = wf * cc[None, :]


def _tc_wf(d2r, w1T_i, b1c_i, w2T_i, b2c_i):
    return pl.pallas_call(
        _wf_body,
        grid=(_K, _E // _BE),
        in_specs=[
            pl.BlockSpec((1, 1, _BE), lambda k, e: (k, 0, e)),
            pl.BlockSpec((_FH, _G), lambda k, e: (0, 0)),
            pl.BlockSpec((_FH, 1), lambda k, e: (0, 0)),
            pl.BlockSpec((_FH, _FH), lambda k, e: (0, 0)),
            pl.BlockSpec((_FH, 1), lambda k, e: (0, 0)),
        ],
        out_specs=pl.BlockSpec((_FH, 1, _BE), lambda k, e: (k, 0, e)),
        out_shape=jax.ShapeDtypeStruct((_K * _FH, 1, _E), jnp.float32),
    )(d2r, w1T_i, b1c_i, w2T_i, b2c_i)


def _embed_body(zf_ref, embT_ref, out_ref):
    zrow = zf_ref[0, 0, :]
    ids = lax.broadcasted_iota(jnp.int32, (100, 1), 0).astype(jnp.float32)
    oh = (zrow[None, :] == ids).astype(jnp.float32)
    out_ref[...] = jnp.dot(embT_ref[...], oh,
                           preferred_element_type=jnp.float32)


def _tc_embed(zf3, embT):
    return pl.pallas_call(
        _embed_body,
        grid=(_NP // _BN0,),
        in_specs=[
            pl.BlockSpec((1, 1, _BN0), lambda b: (b, 0, 0)),
            pl.BlockSpec((_H, 100), lambda b: (0, 0)),
        ],
        out_specs=pl.BlockSpec((_H, _BN0), lambda b: (0, b)),
        out_shape=jax.ShapeDtypeStruct((_H, _NP), jnp.float32),
    )(zf3, embT)


def _x1_body(hT_ref, w_ref, out_ref):
    out_ref[:, 0, :] = jnp.dot(w_ref[...], hT_ref[0],
                               preferred_element_type=jnp.float32)


def _tc_x1(hT, l1T_i):
    return pl.pallas_call(
        _x1_body,
        grid=(_K, _NP // _BN),
        in_specs=[
            pl.BlockSpec((1, _H, _BN), lambda k, n: (k, 0, n)),
            pl.BlockSpec((_FH, _H), lambda k, n: (0, 0)),
        ],
        out_specs=pl.BlockSpec((_FH, 1, _BN), lambda k, n: (k, 0, n)),
        out_shape=jax.ShapeDtypeStruct((_K * _FH, 1, _NP), jnp.float32),
    )(hT, l1T_i)


def _upd_body(aggT_ref, hT_ref, w2T_ref, b2_ref, wT_ref, b_ref, out_ref):
    a = _ssp(jnp.dot(w2T_ref[...], aggT_ref[:, 0, :],
                     preferred_element_type=jnp.float32) + b2_ref[...])
    x2 = jnp.dot(wT_ref[...], a,
                 preferred_element_type=jnp.float32) + b_ref[...]
    out_ref[0] = hT_ref[0] + x2


def _tc_update(aggT, hT, l2T_i, b2c_i, lT_i, bc_i):
    return pl.pallas_call(
        _upd_body,
        grid=(_K, _NP // _BN),
        in_specs=[
            pl.BlockSpec((_FH, 1, _BN), lambda k, n: (k, 0, n)),
            pl.BlockSpec((1, _H, _BN), lambda k, n: (k, 0, n)),
            pl.BlockSpec((_H, _FH), lambda k, n: (0, 0)),
            pl.BlockSpec((_H, 1), lambda k, n: (0, 0)),
            pl.BlockSpec((_H, _H), lambda k, n: (0, 0)),
            pl.BlockSpec((_H, 1), lambda k, n: (0, 0)),
        ],
        out_specs=pl.BlockSpec((1, _H, _BN), lambda k, n: (k, 0, n)),
        out_shape=jax.ShapeDtypeStruct((_K, _H, _NP), jnp.float32),
    )(aggT, hT, l2T_i, b2c_i, lT_i, bc_i)


def _mean_body(hT_ref, out_ref):
    x = hT_ref[0]
    msk = lax.broadcasted_iota(jnp.int32, (1, _NP), 1) < _N
    out_ref[0, 0] = jnp.sum(jnp.where(msk, x, 0.0), axis=1) * (1.0 / _N)


def _tc_mean(hT):
    return pl.pallas_call(
        _mean_body,
        grid=(_K,),
        in_specs=[pl.BlockSpec((1, _H, _NP), lambda k: (k, 0, 0))],
        out_specs=pl.BlockSpec((1, 1, _H), lambda k: (k, 0, 0)),
        out_shape=jax.ShapeDtypeStruct((_K, 1, _H), jnp.float32),
    )(hT)


def _head_body(r_ref, w1_ref, b1_ref, w2T_ref, b2_ref,
               cw1_ref, cb1_ref, cw2_ref, cb2_ref, out_ref):
    r = r_ref[...]
    t = jnp.tanh(jnp.dot(r, w1_ref[...],
                         preferred_element_type=jnp.float32) + b1_ref[...])
    sc = jnp.sum(t * w2T_ref[...], axis=1, keepdims=True) + b2_ref[...]
    m = jnp.max(sc, axis=0, keepdims=True)
    e = jnp.exp(sc - m)
    w = e / jnp.sum(e, axis=0, keepdims=True)
    fused = jnp.sum(w * r, axis=0, keepdims=True)
    hid = jnp.maximum(
        jnp.dot(fused, cw1_ref[...],
                preferred_element_type=jnp.float32) + cb1_ref[...], 0.0)
    out_ref[...] = jnp.dot(hid, cw2_ref[...],
                           preferred_element_type=jnp.float32) + cb2_ref[...]


def _tc_head(reprs, attn_w1, attn_b1r, attn_w2T, attn_b2r,
             cls_w1, cls_b1r, cls_w2, cls_b2r):
    return pl.pallas_call(
        _head_body,
        out_shape=jax.ShapeDtypeStruct((1, _T), jnp.float32),
    )(reprs, attn_w1, attn_b1r, attn_w2T, attn_b2r,
      cls_w1, cls_b1r, cls_w2, cls_b2r)


def kernel(z, pos, edge_index, emb, mlp_w1, mlp_b1, mlp_w2, mlp_b2,
           lin1_w, lin2_w, lin2_b, lin_w, lin_b,
           attn_w1, attn_b1, attn_w2, attn_b2,
           cls_w1, cls_b1, cls_w2, cls_b2):
    posT = jnp.pad(jnp.transpose(pos, (0, 2, 1)),
                   ((0, 0), (0, 0), (0, _NP - _N))).reshape(-1)
    eidx = edge_index.astype(jnp.int32).reshape(-1)

    d2 = _sc_dist(posT, eidx)

    d2r = d2.reshape(_K, 1, _E)

    zf3 = jnp.pad(z.astype(jnp.float32),
                  (0, _NP - _N)).reshape(_NP // _BN0, 1, _BN0)
    hT0 = _tc_embed(zf3, emb.T)
    hT = jnp.broadcast_to(hT0[None], (_K, _H, _NP))

    for i in range(_NI):
        wfT_i = _tc_wf(d2r, jnp.transpose(mlp_w1[i]), mlp_b1[i][:, None],
                       jnp.transpose(mlp_w2[i]), mlp_b2[i][:, None])
        x1T = _tc_x1(hT, jnp.transpose(lin1_w[i]))
        aggT = _sc_scatter(x1T, wfT_i, eidx)
        hT = _tc_update(aggT, hT,
                        jnp.transpose(lin2_w[i]), lin2_b[i][:, None],
                        jnp.transpose(lin_w[i]), lin_b[i][:, None])

    reprs = _tc_mean(hT).reshape(_K, _H)
    out = _tc_head(reprs, attn_w1, attn_b1.reshape(1, _FH),
                   jnp.transpose(attn_w2), attn_b2.reshape(1, 1),
                   cls_w1, cls_b1.reshape(1, _H),
                   cls_w2, cls_b2.reshape(1, _T))
    return out.reshape(_T)


# speedup vs baseline: 8.0388x; 1.0027x over previous
import functools
import math

import jax
import jax.numpy as jnp
from jax import lax
from jax.experimental import pallas as pl
from jax.experimental.pallas import tpu as pltpu
from jax.experimental.pallas import tpu_sc as plsc

_K = 4
_N = 10000
_NP = 10240
_E = 160000
_H = 128
_FH = 64
_G = 50
_NI = 3
_CUTOFF = 10.0
_T = 12

_NTILES = 32
_CE = 6400
_CED = 4000
_BE = 16000
_BN = 2048
_BN0 = 1024


def _ssp(x):
    return (jnp.maximum(x, 0.0) + jnp.log1p(jnp.exp(-jnp.abs(x)))
            - math.log(2.0))


def _sc_dist_body(posT_hbm, eidx_hbm, d2_hbm, posb, srcb, dstb, outb):
    c = lax.axis_index("c")
    s = lax.axis_index("s")
    wid = s * 2 + c
    k = wid // 8
    ebase = (wid % 8) * (_E // 8)
    for r in range(3):
        pltpu.sync_copy(posT_hbm.at[pl.ds((k * 3 + r) * _NP, _NP)],
                        posb.at[pl.ds(r * _NP, _NP)])

    def chunk(ci, carry):
        off = ebase + ci * _CED
        pltpu.sync_copy(eidx_hbm.at[pl.ds(k * 2 * _E + off, _CED)], srcb)
        pltpu.sync_copy(eidx_hbm.at[pl.ds((k * 2 + 1) * _E + off, _CED)], dstb)

        @plsc.parallel_loop(0, _CED // 16, unroll=8)
        def inner(j):
            sv = srcb[pl.ds(j * 16, 16)]
            dv = dstb[pl.ds(j * 16, 16)]
            dx = (plsc.load_gather(posb, [sv])
                  - plsc.load_gather(posb, [dv]))
            dy = (plsc.load_gather(posb, [sv + _NP])
                  - plsc.load_gather(posb, [dv + _NP]))
            dz = (plsc.load_gather(posb, [sv + 2 * _NP])
                  - plsc.load_gather(posb, [dv + 2 * _NP]))
            outb[pl.ds(j * 16, 16)] = dx * dx + dy * dy + dz * dz
        pltpu.sync_copy(outb, d2_hbm.at[pl.ds(k * _E + off, _CED)])
        return carry

    lax.fori_loop(0, (_E // 8) // _CED, chunk, 0)


def _sc_dist(posT, eidx):
    mesh = plsc.VectorSubcoreMesh(core_axis_name="c", subcore_axis_name="s")
    return pl.kernel(
        _sc_dist_body,
        out_type=jax.ShapeDtypeStruct((_K * _E,), jnp.float32),
        mesh=mesh,
        compiler_params=pltpu.CompilerParams(needs_layout_passes=False),
        scratch_types=[
            pltpu.VMEM((3 * _NP,), jnp.float32),
            pltpu.VMEM((_CED,), jnp.int32),
            pltpu.VMEM((_CED,), jnp.int32),
            pltpu.VMEM((_CED,), jnp.float32),
        ],
    )(posT, eidx)


_NCH = _E // _CE


def _sc_scatter_body(x1T_hbm, wfT_hbm, eidx_hbm, aggT_hbm,
                     x1b, aggb,
                     srcA, dstA, wfA0, wfA1,
                     srcB, dstB, wfB0, wfB1, semA, semB):
    c = lax.axis_index("c")
    s = lax.axis_index("s")
    wid = s * 2 + c
    f0 = wid * 2
    zero16 = jnp.zeros((16,), jnp.float32)

    for k in range(_K):
        ebase = k * 2 * _E
        frow = k * _FH + f0
        nrow = frow

        def start(ci, srcX, dstX, wfX0, wfX1, sem):
            off = ci * _CE
            pltpu.async_copy(eidx_hbm.at[pl.ds(ebase + off, _CE)], srcX, sem)
            pltpu.async_copy(eidx_hbm.at[pl.ds(ebase + _E + off, _CE)],
                             dstX, sem)
            pltpu.async_copy(wfT_hbm.at[frow, 0, pl.ds(off, _CE)], wfX0,
                             sem)
            pltpu.async_copy(wfT_hbm.at[frow + 1, 0, pl.ds(off, _CE)],
                             wfX1, sem)

        def drain(ci, srcX, dstX, wfX0, wfX1, sem):
            off = ci * _CE
            pltpu.make_async_copy(eidx_hbm.at[pl.ds(ebase + off, _CE)],
                                  srcX, sem).wait()
            pltpu.make_async_copy(eidx_hbm.at[pl.ds(ebase + _E + off, _CE)],
                                  dstX, sem).wait()
            pltpu.make_async_copy(wfT_hbm.at[frow, 0, pl.ds(off, _CE)],
                                  wfX0, sem).wait()
            pltpu.make_async_copy(wfT_hbm.at[frow + 1, 0, pl.ds(off, _CE)],
                                  wfX1, sem).wait()

        def compute(srcX, dstX, wfX0, wfX1):
            @plsc.parallel_loop(0, _CE // 16, unroll=16)
            def inner(j):
                sv = srcX[pl.ds(j * 16, 16)]
                dv = dstX[pl.ds(j * 16, 16)]
                g0 = plsc.load_gather(x1b, [sv])
                g1 = plsc.load_gather(x1b, [sv + _NP])
                m0 = g0 * wfX0[pl.ds(j * 16, 16)]
                m1 = g1 * wfX1[pl.ds(j * 16, 16)]
                plsc.addupdate_scatter(aggb, [dv], m0)
                plsc.addupdate_scatter(aggb, [dv + _NP], m1)

        start(0, srcA, dstA, wfA0, wfA1, semA)

        @plsc.parallel_loop(0, (2 * _NP) // 16, unroll=8)
        def zbody(j):
            aggb[pl.ds(j * 16, 16)] = zero16

        pltpu.sync_copy(x1T_hbm.at[nrow, 0], x1b.at[pl.ds(0, _NP)])
        pltpu.sync_copy(x1T_hbm.at[nrow + 1, 0], x1b.at[pl.ds(_NP, _NP)])

        def pair(ci2, carry):
            ciA = 2 * ci2
            start(ciA + 1, srcB, dstB, wfB0, wfB1, semB)
            drain(ciA, srcA, dstA, wfA0, wfA1, semA)
            compute(srcA, dstA, wfA0, wfA1)
            start(ciA + 2, srcA, dstA, wfA0, wfA1, semA)
            drain(ciA + 1, srcB, dstB, wfB0, wfB1, semB)
            compute(srcB, dstB, wfB0, wfB1)
            return carry

        lax.fori_loop(0, _NCH // 2, pair, 0)
        drain(_NCH - 1, srcA, dstA, wfA0, wfA1, semA)
        compute(srcA, dstA, wfA0, wfA1)

        pltpu.sync_copy(aggb.at[pl.ds(0, _NP)], aggT_hbm.at[nrow, 0])
        pltpu.sync_copy(aggb.at[pl.ds(_NP, _NP)], aggT_hbm.at[nrow + 1, 0])


def _sc_scatter(x1T, wfT_i, eidx):
    mesh = plsc.VectorSubcoreMesh(core_axis_name="c", subcore_axis_name="s")
    return pl.kernel(
        _sc_scatter_body,
        out_type=jax.ShapeDtypeStruct((_K * _FH, 1, _NP), jnp.float32),
        mesh=mesh,
        compiler_params=pltpu.CompilerParams(needs_layout_passes=False),
        scratch_types=[
            pltpu.VMEM((2 * _NP,), jnp.float32),
            pltpu.VMEM((2 * _NP,), jnp.float32),
            pltpu.VMEM((_CE,), jnp.int32),
            pltpu.VMEM((_CE,), jnp.int32),
            pltpu.VMEM((_CE,), jnp.float32),
            pltpu.VMEM((_CE,), jnp.float32),
            pltpu.VMEM((_CE,), jnp.int32),
            pltpu.VMEM((_CE,), jnp.int32),
            pltpu.VMEM((_CE,), jnp.float32),
            pltpu.VMEM((_CE,), jnp.float32),
            pltpu.SemaphoreType.DMA,
            pltpu.SemaphoreType.DMA,
        ],
    )(x1T, wfT_i, eidx)


def _wf_body(d2_ref, w1T_ref, b1_ref, w2T_ref, b2_ref, out_ref):
    d2 = d2_ref[0, 0, :]
    dist = jnp.sqrt(d2)
    delta = _CUTOFF / (_G - 1)
    coeff = -0.5 / (delta * delta)
    offs = lax.broadcasted_iota(jnp.int32, (_G, 1), 0).astype(jnp.float32) * delta
    diff = dist[None, :] - offs
    eaT = jnp.exp(coeff * (diff * diff))
    a = _ssp(jnp.dot(w1T_ref[...], eaT,
                     preferred_element_type=jnp.float32) + b1_ref[...])
    wf = (jnp.dot(w2T_ref[...], a, preferred_element_type=jnp.float32)
          + b2_ref[...])
    cc = 0.5 * (jnp.cos(dist * (math.pi / _CUTOFF)) + 1.0)
    out_ref[:, 0, :] = wf * cc[None, :]


def _tc_wf(d2r, w1T_i, b1c_i, w2T_i, b2c_i):
    return pl.pallas_call(
        _wf_body,
        grid=(_K, _E // _BE),
        in_specs=[
            pl.BlockSpec((1, 1, _BE), lambda k, e: (k, 0, e)),
            pl.BlockSpec((_FH, _G), lambda k, e: (0, 0)),
            pl.BlockSpec((_FH, 1), lambda k, e: (0, 0)),
            pl.BlockSpec((_FH, _FH), lambda k, e: (0, 0)),
            pl.BlockSpec((_FH, 1), lambda k, e: (0, 0)),
        ],
        out_specs=pl.BlockSpec((_FH, 1, _BE), lambda k, e: (k, 0, e)),
        out_shape=jax.ShapeDtypeStruct((_K * _FH, 1, _E), jnp.float32),
    )(d2r, w1T_i, b1c_i, w2T_i, b2c_i)


def _embed_body(zf_ref, embT_ref, out_ref):
    zrow = zf_ref[0, 0, :]
    ids = lax.broadcasted_iota(jnp.int32, (100, 1), 0).astype(jnp.float32)
    oh = (zrow[None, :] == ids).astype(jnp.float32)
    out_ref[...] = jnp.dot(embT_ref[...], oh,
                           preferred_element_type=jnp.float32)


def _tc_embed(zf3, embT):
    return pl.pallas_call(
        _embed_body,
        grid=(_NP // _BN0,),
        in_specs=[
            pl.BlockSpec((1, 1, _BN0), lambda b: (b, 0, 0)),
            pl.BlockSpec((_H, 100), lambda b: (0, 0)),
        ],
        out_specs=pl.BlockSpec((_H, _BN0), lambda b: (0, b)),
        out_shape=jax.ShapeDtypeStruct((_H, _NP), jnp.float32),
    )(zf3, embT)


def _x1_body(hT_ref, w_ref, out_ref):
    out_ref[:, 0, :] = jnp.dot(w_ref[...], hT_ref[0],
                               preferred_element_type=jnp.float32)


def _tc_x1(hT, l1T_i):
    return pl.pallas_call(
        _x1_body,
        grid=(_K, _NP // _BN),
        in_specs=[
            pl.BlockSpec((1, _H, _BN), lambda k, n: (k, 0, n)),
            pl.BlockSpec((_FH, _H), lambda k, n: (0, 0)),
        ],
        out_specs=pl.BlockSpec((_FH, 1, _BN), lambda k, n: (k, 0, n)),
        out_shape=jax.ShapeDtypeStruct((_K * _FH, 1, _NP), jnp.float32),
    )(hT, l1T_i)


def _upd_body(aggT_ref, hT_ref, w2T_ref, b2_ref, wT_ref, b_ref, out_ref):
    a = _ssp(jnp.dot(w2T_ref[...], aggT_ref[:, 0, :],
                     preferred_element_type=jnp.float32) + b2_ref[...])
    x2 = jnp.dot(wT_ref[...], a,
                 preferred_element_type=jnp.float32) + b_ref[...]
    out_ref[0] = hT_ref[0] + x2


def _tc_update(aggT, hT, l2T_i, b2c_i, lT_i, bc_i):
    return pl.pallas_call(
        _upd_body,
        grid=(_K, _NP // _BN),
        in_specs=[
            pl.BlockSpec((_FH, 1, _BN), lambda k, n: (k, 0, n)),
            pl.BlockSpec((1, _H, _BN), lambda k, n: (k, 0, n)),
            pl.BlockSpec((_H, _FH), lambda k, n: (0, 0)),
            pl.BlockSpec((_H, 1), lambda k, n: (0, 0)),
            pl.BlockSpec((_H, _H), lambda k, n: (0, 0)),
            pl.BlockSpec((_H, 1), lambda k, n: (0, 0)),
        ],
        out_specs=pl.BlockSpec((1, _H, _BN), lambda k, n: (k, 0, n)),
        out_shape=jax.ShapeDtypeStruct((_K, _H, _NP), jnp.float32),
    )(aggT, hT, l2T_i, b2c_i, lT_i, bc_i)


def _mean_body(hT_ref, out_ref):
    x = hT_ref[0]
    msk = lax.broadcasted_iota(jnp.int32, (1, _NP), 1) < _N
    out_ref[0, 0] = jnp.sum(jnp.where(msk, x, 0.0), axis=1) * (1.0 / _N)


def _tc_mean(hT):
    return pl.pallas_call(
        _mean_body,
        grid=(_K,),
        in_specs=[pl.BlockSpec((1, _H, _NP), lambda k: (k, 0, 0))],
        out_specs=pl.BlockSpec((1, 1, _H), lambda k: (k, 0, 0)),
        out_shape=jax.ShapeDtypeStruct((_K, 1, _H), jnp.float32),
    )(hT)


def _head_body(r_ref, w1_ref, b1_ref, w2T_ref, b2_ref,
               cw1_ref, cb1_ref, cw2_ref, cb2_ref, out_ref):
    r = r_ref[...]
    t = jnp.tanh(jnp.dot(r, w1_ref[...],
                         preferred_element_type=jnp.float32) + b1_ref[...])
    sc = jnp.sum(t * w2T_ref[...], axis=1, keepdims=True) + b2_ref[...]
    m = jnp.max(sc, axis=0, keepdims=True)
    e = jnp.exp(sc - m)
    w = e / jnp.sum(e, axis=0, keepdims=True)
    fused = jnp.sum(w * r, axis=0, keepdims=True)
    hid = jnp.maximum(
        jnp.dot(fused, cw1_ref[...],
                preferred_element_type=jnp.float32) + cb1_ref[...], 0.0)
    out_ref[...] = jnp.dot(hid, cw2_ref[...],
                           preferred_element_type=jnp.float32) + cb2_ref[...]


def _tc_head(reprs, attn_w1, attn_b1r, attn_w2T, attn_b2r,
             cls_w1, cls_b1r, cls_w2, cls_b2r):
    return pl.pallas_call(
        _head_body,
        out_shape=jax.ShapeDtypeStruct((1, _T), jnp.float32),
    )(reprs, attn_w1, attn_b1r, attn_w2T, attn_b2r,
      cls_w1, cls_b1r, cls_w2, cls_b2r)


def kernel(z, pos, edge_index, emb, mlp_w1, mlp_b1, mlp_w2, mlp_b2,
           lin1_w, lin2_w, lin2_b, lin_w, lin_b,
           attn_w1, attn_b1, attn_w2, attn_b2,
           cls_w1, cls_b1, cls_w2, cls_b2):
    posT = jnp.pad(jnp.transpose(pos, (0, 2, 1)),
                   ((0, 0), (0, 0), (0, _NP - _N))).reshape(-1)
    eidx = edge_index.astype(jnp.int32).reshape(-1)

    d2 = _sc_dist(posT, eidx)

    d2r = d2.reshape(_K, 1, _E)

    zf3 = jnp.pad(z.astype(jnp.float32),
                  (0, _NP - _N)).reshape(_NP // _BN0, 1, _BN0)
    hT0 = _tc_embed(zf3, emb.T)
    hT = jnp.broadcast_to(hT0[None], (_K, _H, _NP))

    for i in range(_NI):
        wfT_i = _tc_wf(d2r, jnp.transpose(mlp_w1[i]), mlp_b1[i][:, None],
                       jnp.transpose(mlp_w2[i]), mlp_b2[i][:, None])
        x1T = _tc_x1(hT, jnp.transpose(lin1_w[i]))
        aggT = _sc_scatter(x1T, wfT_i, eidx)
        hT = _tc_update(aggT, hT,
                        jnp.transpose(lin2_w[i]), lin2_b[i][:, None],
                        jnp.transpose(lin_w[i]), lin_b[i][:, None])

    reprs = _tc_mean(hT).reshape(_K, _H)
    out = _tc_head(reprs, attn_w1, attn_b1.reshape(1, _FH),
                   jnp.transpose(attn_w2), attn_b2.reshape(1, 1),
                   cls_w1, cls_b1.reshape(1, _H),
                   cls_w2, cls_b2.reshape(1, _T))
    return out.reshape(_T)


# speedup vs baseline: 8.0486x; 1.0012x over previous
import math

import jax
import jax.numpy as jnp
from jax import lax
from jax.experimental import pallas as pl
from jax.experimental.pallas import tpu as pltpu
from jax.experimental.pallas import tpu_sc as plsc

_K = 4
_N = 10000
_NP = 10240
_E = 160000
_H = 128
_FH = 64
_G = 50
_NI = 3
_CUTOFF = 10.0
_T = 12

_NTILES = 32
_CE = 6400
_CED = 4000
_BE = 16000
_BN = 2048
_BN0 = 1024


def _ssp(x):
    return (jnp.maximum(x, 0.0) + jnp.log1p(jnp.exp(-jnp.abs(x)))
            - math.log(2.0))


def _sc_dist_body(posT_hbm, eidx_hbm, d2_hbm, posb, srcb, dstb, outb):
    c = lax.axis_index("c")
    s = lax.axis_index("s")
    wid = s * 2 + c
    k = wid // 8
    ebase = (wid % 8) * (_E // 8)
    for r in range(3):
        pltpu.sync_copy(posT_hbm.at[pl.ds((k * 3 + r) * _NP, _NP)],
                        posb.at[pl.ds(r * _NP, _NP)])

    def chunk(ci, carry):
        off = ebase + ci * _CED
        pltpu.sync_copy(eidx_hbm.at[pl.ds(k * 2 * _E + off, _CED)], srcb)
        pltpu.sync_copy(eidx_hbm.at[pl.ds((k * 2 + 1) * _E + off, _CED)], dstb)

        @plsc.parallel_loop(0, _CED // 16, unroll=8)
        def inner(j):
            sv = srcb[pl.ds(j * 16, 16)]
            dv = dstb[pl.ds(j * 16, 16)]
            dx = (plsc.load_gather(posb, [sv])
                  - plsc.load_gather(posb, [dv]))
            dy = (plsc.load_gather(posb, [sv + _NP])
                  - plsc.load_gather(posb, [dv + _NP]))
            dz = (plsc.load_gather(posb, [sv + 2 * _NP])
                  - plsc.load_gather(posb, [dv + 2 * _NP]))
            outb[pl.ds(j * 16, 16)] = dx * dx + dy * dy + dz * dz
        pltpu.sync_copy(outb, d2_hbm.at[pl.ds(k * _E + off, _CED)])
        return carry

    lax.fori_loop(0, (_E // 8) // _CED, chunk, 0)


def _sc_dist(posT, eidx):
    mesh = plsc.VectorSubcoreMesh(core_axis_name="c", subcore_axis_name="s")
    return pl.kernel(
        _sc_dist_body,
        out_type=jax.ShapeDtypeStruct((_K * _E,), jnp.float32),
        mesh=mesh,
        compiler_params=pltpu.CompilerParams(needs_layout_passes=False),
        scratch_types=[
            pltpu.VMEM((3 * _NP,), jnp.float32),
            pltpu.VMEM((_CED,), jnp.int32),
            pltpu.VMEM((_CED,), jnp.int32),
            pltpu.VMEM((_CED,), jnp.float32),
        ],
    )(posT, eidx)


_NCH = _E // _CE


def _sc_scatter_body(x1T_hbm, wfT_hbm, eidx_hbm, aggT_hbm,
                     x1b, aggb,
                     srcA, dstA, wfA0, wfA1,
                     srcB, dstB, wfB0, wfB1, semA, semB):
    c = lax.axis_index("c")
    s = lax.axis_index("s")
    wid = s * 2 + c
    f0 = wid * 2
    zero16 = jnp.zeros((16,), jnp.float32)

    for k in range(_K):
        ebase = k * 2 * _E
        frow = k * _FH + f0
        nrow = frow

        def start(ci, srcX, dstX, wfX0, wfX1, sem):
            off = ci * _CE
            pltpu.async_copy(eidx_hbm.at[pl.ds(ebase + off, _CE)], srcX, sem)
            pltpu.async_copy(eidx_hbm.at[pl.ds(ebase + _E + off, _CE)],
                             dstX, sem)
            pltpu.async_copy(wfT_hbm.at[frow, 0, pl.ds(off, _CE)], wfX0,
                             sem)
            pltpu.async_copy(wfT_hbm.at[frow + 1, 0, pl.ds(off, _CE)],
                             wfX1, sem)

        def drain(ci, srcX, dstX, wfX0, wfX1, sem):
            off = ci * _CE
            pltpu.make_async_copy(eidx_hbm.at[pl.ds(ebase + off, _CE)],
                                  srcX, sem).wait()
            pltpu.make_async_copy(eidx_hbm.at[pl.ds(ebase + _E + off, _CE)],
                                  dstX, sem).wait()
            pltpu.make_async_copy(wfT_hbm.at[frow, 0, pl.ds(off, _CE)],
                                  wfX0, sem).wait()
            pltpu.make_async_copy(wfT_hbm.at[frow + 1, 0, pl.ds(off, _CE)],
                                  wfX1, sem).wait()

        def compute(srcX, dstX, wfX0, wfX1):
            @plsc.parallel_loop(0, _CE // 16, unroll=16)
            def inner(j):
                sv = srcX[pl.ds(j * 16, 16)]
                dv = dstX[pl.ds(j * 16, 16)]
                g0 = plsc.load_gather(x1b, [sv])
                g1 = plsc.load_gather(x1b, [sv + _NP])
                m0 = g0 * wfX0[pl.ds(j * 16, 16)]
                m1 = g1 * wfX1[pl.ds(j * 16, 16)]
                plsc.addupdate_scatter(aggb, [dv], m0)
                plsc.addupdate_scatter(aggb, [dv + _NP], m1)

        start(0, srcA, dstA, wfA0, wfA1, semA)

        @plsc.parallel_loop(0, (2 * _NP) // 16, unroll=8)
        def zbody(j):
            aggb[pl.ds(j * 16, 16)] = zero16

        pltpu.sync_copy(x1T_hbm.at[nrow, 0], x1b.at[pl.ds(0, _NP)])
        pltpu.sync_copy(x1T_hbm.at[nrow + 1, 0], x1b.at[pl.ds(_NP, _NP)])

        def pair(ci2, carry):
            ciA = 2 * ci2
            start(ciA + 1, srcB, dstB, wfB0, wfB1, semB)
            drain(ciA, srcA, dstA, wfA0, wfA1, semA)
            compute(srcA, dstA, wfA0, wfA1)
            start(ciA + 2, srcA, dstA, wfA0, wfA1, semA)
            drain(ciA + 1, srcB, dstB, wfB0, wfB1, semB)
            compute(srcB, dstB, wfB0, wfB1)
            return carry

        lax.fori_loop(0, _NCH // 2, pair, 0)
        drain(_NCH - 1, srcA, dstA, wfA0, wfA1, semA)
        compute(srcA, dstA, wfA0, wfA1)

        pltpu.sync_copy(aggb.at[pl.ds(0, _NP)], aggT_hbm.at[nrow, 0])
        pltpu.sync_copy(aggb.at[pl.ds(_NP, _NP)], aggT_hbm.at[nrow + 1, 0])


def _sc_scatter(x1T, wfT_i, eidx):
    mesh = plsc.VectorSubcoreMesh(core_axis_name="c", subcore_axis_name="s")
    return pl.kernel(
        _sc_scatter_body,
        out_type=jax.ShapeDtypeStruct((_K * _FH, 1, _NP), jnp.float32),
        mesh=mesh,
        compiler_params=pltpu.CompilerParams(needs_layout_passes=False),
        scratch_types=[
            pltpu.VMEM((2 * _NP,), jnp.float32),
            pltpu.VMEM((2 * _NP,), jnp.float32),
            pltpu.VMEM((_CE,), jnp.int32),
            pltpu.VMEM((_CE,), jnp.int32),
            pltpu.VMEM((_CE,), jnp.float32),
            pltpu.VMEM((_CE,), jnp.float32),
            pltpu.VMEM((_CE,), jnp.int32),
            pltpu.VMEM((_CE,), jnp.int32),
            pltpu.VMEM((_CE,), jnp.float32),
            pltpu.VMEM((_CE,), jnp.float32),
            pltpu.SemaphoreType.DMA,
            pltpu.SemaphoreType.DMA,
        ],
    )(x1T, wfT_i, eidx)


def _wf_body(d2_ref, w1T_ref, b1_ref, w2T_ref, b2_ref, out_ref):
    d2 = d2_ref[0, 0, :]
    dist = jnp.sqrt(d2)
    delta = _CUTOFF / (_G - 1)
    coeff = -0.5 / (delta * delta)
    offs = lax.broadcasted_iota(jnp.int32, (_G, 1), 0).astype(jnp.float32) * delta
    diff = dist[None, :] - offs
    eaT = jnp.exp(coeff * (diff * diff))
    a = _ssp(jnp.dot(w1T_ref[...], eaT,
                     preferred_element_type=jnp.float32) + b1_ref[...])
    wf = (jnp.dot(w2T_ref[...], a, preferred_element_type=jnp.float32)
          + b2_ref[...])
    cc = 0.5 * (jnp.cos(dist * (math.pi / _CUTOFF)) + 1.0)
    out_ref[:, 0, :] = wf * cc[None, :]


def _tc_wf(d2r, w1T_i, b1c_i, w2T_i, b2c_i):
    return pl.pallas_call(
        _wf_body,
        grid=(_K, _E // _BE),
        in_specs=[
            pl.BlockSpec((1, 1, _BE), lambda k, e: (k, 0, e)),
            pl.BlockSpec((_FH, _G), lambda k, e: (0, 0)),
            pl.BlockSpec((_FH, 1), lambda k, e: (0, 0)),
            pl.BlockSpec((_FH, _FH), lambda k, e: (0, 0)),
            pl.BlockSpec((_FH, 1), lambda k, e: (0, 0)),
        ],
        out_specs=pl.BlockSpec((_FH, 1, _BE), lambda k, e: (k, 0, e)),
        out_shape=jax.ShapeDtypeStruct((_K * _FH, 1, _E), jnp.float32),
    )(d2r, w1T_i, b1c_i, w2T_i, b2c_i)


def _embed_body(zf_ref, embT_ref, out_ref):
    zrow = zf_ref[0, 0, :]
    ids = lax.broadcasted_iota(jnp.int32, (100, 1), 0).astype(jnp.float32)
    oh = (zrow[None, :] == ids).astype(jnp.float32)
    out_ref[...] = jnp.dot(embT_ref[...], oh,
                           preferred_element_type=jnp.float32)


def _tc_embed(zf3, embT):
    return pl.pallas_call(
        _embed_body,
        grid=(_NP // _BN0,),
        in_specs=[
            pl.BlockSpec((1, 1, _BN0), lambda b: (b, 0, 0)),
            pl.BlockSpec((_H, 100), lambda b: (0, 0)),
        ],
        out_specs=pl.BlockSpec((_H, _BN0), lambda b: (0, b)),
        out_shape=jax.ShapeDtypeStruct((_H, _NP), jnp.float32),
    )(zf3, embT)


def _x1_body(hT_ref, w_ref, out_ref):
    out_ref[:, 0, :] = jnp.dot(w_ref[...], hT_ref[0],
                               preferred_element_type=jnp.float32)


def _tc_x1(hT, l1T_i):
    return pl.pallas_call(
        _x1_body,
        grid=(_K, _NP // _BN),
        in_specs=[
            pl.BlockSpec((1, _H, _BN), lambda k, n: (k, 0, n)),
            pl.BlockSpec((_FH, _H), lambda k, n: (0, 0)),
        ],
        out_specs=pl.BlockSpec((_FH, 1, _BN), lambda k, n: (k, 0, n)),
        out_shape=jax.ShapeDtypeStruct((_K * _FH, 1, _NP), jnp.float32),
    )(hT, l1T_i)


def _upd_body(aggT_ref, hT_ref, w2T_ref, b2_ref, wT_ref, b_ref, out_ref):
    a = _ssp(jnp.dot(w2T_ref[...], aggT_ref[:, 0, :],
                     preferred_element_type=jnp.float32) + b2_ref[...])
    x2 = jnp.dot(wT_ref[...], a,
                 preferred_element_type=jnp.float32) + b_ref[...]
    out_ref[0] = hT_ref[0] + x2


def _tc_update(aggT, hT, l2T_i, b2c_i, lT_i, bc_i):
    return pl.pallas_call(
        _upd_body,
        grid=(_K, _NP // _BN),
        in_specs=[
            pl.BlockSpec((_FH, 1, _BN), lambda k, n: (k, 0, n)),
            pl.BlockSpec((1, _H, _BN), lambda k, n: (k, 0, n)),
            pl.BlockSpec((_H, _FH), lambda k, n: (0, 0)),
            pl.BlockSpec((_H, 1), lambda k, n: (0, 0)),
            pl.BlockSpec((_H, _H), lambda k, n: (0, 0)),
            pl.BlockSpec((_H, 1), lambda k, n: (0, 0)),
        ],
        out_specs=pl.BlockSpec((1, _H, _BN), lambda k, n: (k, 0, n)),
        out_shape=jax.ShapeDtypeStruct((_K, _H, _NP), jnp.float32),
    )(aggT, hT, l2T_i, b2c_i, lT_i, bc_i)


def _mean_body(hT_ref, out_ref):
    x = hT_ref[0]
    msk = lax.broadcasted_iota(jnp.int32, (1, _NP), 1) < _N
    out_ref[0, 0] = jnp.sum(jnp.where(msk, x, 0.0), axis=1) * (1.0 / _N)


def _tc_mean(hT):
    return pl.pallas_call(
        _mean_body,
        grid=(_K,),
        in_specs=[pl.BlockSpec((1, _H, _NP), lambda k: (k, 0, 0))],
        out_specs=pl.BlockSpec((1, 1, _H), lambda k: (k, 0, 0)),
        out_shape=jax.ShapeDtypeStruct((_K, 1, _H), jnp.float32),
    )(hT)


def _head_body(r_ref, w1_ref, b1_ref, w2T_ref, b2_ref,
               cw1_ref, cb1_ref, cw2_ref, cb2_ref, out_ref):
    r = r_ref[...]
    t = jnp.tanh(jnp.dot(r, w1_ref[...],
                         preferred_element_type=jnp.float32) + b1_ref[...])
    sc = jnp.sum(t * w2T_ref[...], axis=1, keepdims=True) + b2_ref[...]
    m = jnp.max(sc, axis=0, keepdims=True)
    e = jnp.exp(sc - m)
    w = e / jnp.sum(e, axis=0, keepdims=True)
    fused = jnp.sum(w * r, axis=0, keepdims=True)
    hid = jnp.maximum(
        jnp.dot(fused, cw1_ref[...],
                preferred_element_type=jnp.float32) + cb1_ref[...], 0.0)
    out_ref[...] = jnp.dot(hid, cw2_ref[...],
                           preferred_element_type=jnp.float32) + cb2_ref[...]


def _tc_head(reprs, attn_w1, attn_b1r, attn_w2T, attn_b2r,
             cls_w1, cls_b1r, cls_w2, cls_b2r):
    return pl.pallas_call(
        _head_body,
        out_shape=jax.ShapeDtypeStruct((1, _T), jnp.float32),
    )(reprs, attn_w1, attn_b1r, attn_w2T, attn_b2r,
      cls_w1, cls_b1r, cls_w2, cls_b2r)


def kernel(z, pos, edge_index, emb, mlp_w1, mlp_b1, mlp_w2, mlp_b2,
           lin1_w, lin2_w, lin2_b, lin_w, lin_b,
           attn_w1, attn_b1, attn_w2, attn_b2,
           cls_w1, cls_b1, cls_w2, cls_b2):
    posT = jnp.pad(jnp.transpose(pos, (0, 2, 1)),
                   ((0, 0), (0, 0), (0, _NP - _N))).reshape(-1)
    eidx = edge_index.astype(jnp.int32).reshape(-1)

    d2 = _sc_dist(posT, eidx)

    d2r = d2.reshape(_K, 1, _E)

    zf3 = jnp.pad(z.astype(jnp.float32),
                  (0, _NP - _N)).reshape(_NP // _BN0, 1, _BN0)
    hT0 = _tc_embed(zf3, emb.T)
    hT = jnp.broadcast_to(hT0[None], (_K, _H, _NP))

    for i in range(_NI):
        wfT_i = _tc_wf(d2r, jnp.transpose(mlp_w1[i]), mlp_b1[i][:, None],
                       jnp.transpose(mlp_w2[i]), mlp_b2[i][:, None])
        x1T = _tc_x1(hT, jnp.transpose(lin1_w[i]))
        aggT = _sc_scatter(x1T, wfT_i, eidx)
        hT = _tc_update(aggT, hT,
                        jnp.transpose(lin2_w[i]), lin2_b[i][:, None],
                        jnp.transpose(lin_w[i]), lin_b[i][:, None])

    reprs = _tc_mean(hT).reshape(_K, _H)
    out = _tc_head(reprs, attn_w1, attn_b1.reshape(1, _FH),
                   jnp.transpose(attn_w2), attn_b2.reshape(1, 1),
                   cls_w1, cls_b1.reshape(1, _H),
                   cls_w2, cls_b2.reshape(1, _T))
    return out.reshape(_T)
